# Initial kernel scaffold; baseline (speedup 1.0000x reference)
import jax
import jax.numpy as jnp
from jax.experimental import pallas as pl


def kernel(X_in1, edge_index, metadata_in1, W1, b1, W2, b2):
    raise NotImplementedError("write your pallas kernel here")


# speedup vs baseline: 135.7943x; 135.7943x over previous
import functools
import jax
import jax.numpy as jnp
from jax import lax
from jax.experimental import pallas as pl
from jax.experimental.pallas import tpu as pltpu
from jax.experimental.pallas import tpu_sc as plsc

_N = 50000
_E = 800000
_H = 4
_D = 16
_ALPHA = 0.2
_NC = 2
_NS = 16
_NW = _NC * _NS
_EPW = _E // _NW
_CH = 1000
_NCHUNK = _EPW // _CH
_BN = 5000
_NBLK = _N // _BN
_RPT = 3128
_RLAST = _N - 15 * _RPT


def _leaky(v):
    return jnp.maximum(v, _ALPHA * v)


def _hyper_body(x_ref, md_ref, w1_ref, b1_ref,
                wvw_ref, wvb_ref, bfw_ref, bfb_ref, alw_ref, alb_ref,
                arw_ref, arb_ref, w3w_ref, w3b_ref, bow_ref, bob_ref,
                aolw_ref, aolb_ref, aorw_ref, aorb_ref,
                g4_ref, g64_ref, g16_ref, rrep_ref, g4c_ref, ones_ref,
                out_ref):
    h = jnp.tanh(md_ref[...] @ w1_ref[...] + b1_ref[...])
    wvf = h @ wvw_ref[...] + wvb_ref[...]
    bfcf = h @ bfw_ref[...] + bfb_ref[...]
    alf = h @ alw_ref[...] + alb_ref[...]
    arf = h @ arw_ref[...] + arb_ref[...]
    w3f = h @ w3w_ref[...] + w3b_ref[...]
    b_out = h @ bow_ref[...] + bob_ref[...]
    a_l = h @ aolw_ref[...] + aolb_ref[...]
    a_r = h @ aorw_ref[...] + aorb_ref[...]

    g4 = g4_ref[...]
    cl = (wvf * alf) @ g4
    cr = (wvf * arf) @ g4
    wvt = jnp.concatenate([wvf] * 16, axis=1)
    bft = jnp.concatenate([bfcf] * 16, axis=1)
    m_flat = (w3f * wvt) @ g64_ref[...]
    c0 = (w3f * bft) @ g16_ref[...]
    alr = a_l @ rrep_ref[...]
    arr = a_r @ rrep_ref[...]
    g4c = g4c_ref[...]
    ml = (m_flat * alr) @ g4c
    mr = (m_flat * arr) @ g4c
    ones16 = ones_ref[...]
    cl2 = (c0 * a_l) @ ones16
    cr2 = (c0 * a_r) @ ones16

    xv = x_ref[...]
    mx = jnp.max(xv)
    mn = jnp.min(xv)
    ub = jnp.maximum(mx * cl, mn * cl) + jnp.maximum(mx * cr, mn * cr)
    m1 = _leaky(ub)

    z = jnp.zeros((1, 4), jnp.float32)
    out_ref[...] = jnp.concatenate(
        [cl, cr, m1, z, ml, mr, cl2, cr2, z[:, 0:2],
         m_flat, c0, b_out, z], axis=1)


def _run_hyper(x, metadata, w1, b1, w2, b2):
    hd = _H * _D
    o1, o2, o3 = 4 * hd, 4 * hd + _D * hd, 4 * hd + _D * hd + _D
    pieces = [(0, hd), (hd, 2 * hd), (2 * hd, 3 * hd), (3 * hd, 4 * hd),
              (o1, o2), (o2, o3), (o3, o3 + _D), (o3 + _D, o3 + 2 * _D)]
    args = [x, metadata, w1, b1[None]]
    for lo, hi in pieces:
        args.append(w2[:, lo:hi])
        args.append(b2[lo:hi][None])
    i64 = jnp.arange(64)
    i1024 = jnp.arange(1024)
    i16 = jnp.arange(16)
    g4 = (i64[:, None] // 16 == jnp.arange(4)[None]).astype(jnp.float32)
    kh = (i1024 // 64) * 4 + (i1024 % 64) // 16
    g64 = (kh[:, None] == i64[None]).astype(jnp.float32)
    g16 = (i1024[:, None] // 64 == i16[None]).astype(jnp.float32)
    rrep = (i16[:, None] == i64[None] // 4).astype(jnp.float32)
    g4c = (i64[:, None] % 4 == jnp.arange(4)[None]).astype(jnp.float32)
    ones16 = jnp.ones((16, 1), jnp.float32)
    args += [g4, g64, g16, rrep, g4c, ones16]
    cf = pl.pallas_call(
        _hyper_body,
        out_shape=jax.ShapeDtypeStruct((1, 128), jnp.float32),
    )(*args)[0]
    c1 = jnp.stack([jnp.tile(cf[0:4], 4), jnp.tile(cf[4:8], 4),
                    jnp.tile(cf[8:12], 4)])
    m = cf[28:92].reshape(16, 4).T
    row0 = jnp.concatenate([cf[16:24], cf[24:26], jnp.zeros((6,), jnp.float32)])
    c2 = jnp.concatenate([row0[None], m, cf[92:108][None], cf[108:124][None],
                          jnp.zeros((1, 16), jnp.float32)], axis=0)
    return c1, c2


def _edge1_body(src_hbm, dst_hbm, x_hbm, c1_hbm, zeros_hbm, out_hbm,
                srcv, dstv, xsv, xdv, rowsv, cv, acc, sem1, sem2):
    cid = lax.axis_index("c")
    sid = lax.axis_index("s")
    wid = sid * _NC + cid

    pltpu.sync_copy(c1_hbm, cv)

    @pl.when(sid == 0)
    def _zero():
        pltpu.sync_copy(zeros_hbm, acc)

    plsc.subcore_barrier()

    clv = cv[0, :]
    crv = cv[1, :]
    m1v = cv[2, :]
    lane = lax.iota(jnp.int32, 16)
    egrp = lane >> 3
    col = lane & 7
    is_q = col >= 4

    def chunk(j, carry):
        base = wid * _EPW + j * _CH
        pltpu.sync_copy(src_hbm.at[pl.ds(base, _CH)], srcv)
        pltpu.sync_copy(dst_hbm.at[pl.ds(base, _CH)], dstv)
        cp1 = pltpu.async_copy(x_hbm.at[srcv], xsv, sem1)
        cp2 = pltpu.async_copy(x_hbm.at[dstv], xdv, sem2)
        cp1.wait()
        cp2.wait()

        def pair(p, c2):
            pidx = p * 2 + egrp
            xs = plsc.load_gather(xsv, [pidx])
            xd = plsc.load_gather(xdv, [pidx])
            pre = xs * clv + xd * crv
            pz = jnp.exp(jnp.maximum(pre, _ALPHA * pre) - m1v)
            plsc.store_scatter(rowsv, [pidx, col], jnp.where(is_q, pz * xs, pz))
            return c2

        lax.fori_loop(0, _CH // 2, pair, 0, unroll=8)
        pltpu.sync_copy(rowsv, acc.at[dstv], add=True)
        return carry

    lax.fori_loop(0, _NCHUNK, chunk, 0)
    plsc.subcore_barrier()

    @pl.when(sid < 15)
    def _copy_out():
        r0 = sid * _RPT
        pltpu.sync_copy(acc.at[pl.ds(r0, _RPT)],
                        out_hbm.at[cid, pl.ds(r0, _RPT)])

    @pl.when(sid == 15)
    def _copy_last():
        r0 = 15 * _RPT
        pltpu.sync_copy(acc.at[pl.ds(r0, _RLAST)],
                        out_hbm.at[cid, pl.ds(r0, _RLAST)])


_edge1 = functools.partial(
    pl.kernel,
    out_type=jax.ShapeDtypeStruct((_NC, _N, 8), jnp.float32),
    mesh=plsc.VectorSubcoreMesh(core_axis_name="c", subcore_axis_name="s",
                                num_cores=_NC, num_subcores=_NS),
    compiler_params=pltpu.CompilerParams(
        use_tc_tiling_on_sc=False, needs_layout_passes=False),
    scratch_types=[
        pltpu.VMEM((_CH,), jnp.int32),
        pltpu.VMEM((_CH,), jnp.int32),
        pltpu.VMEM((_CH,), jnp.float32),
        pltpu.VMEM((_CH,), jnp.float32),
        pltpu.VMEM((_CH, 8), jnp.float32),
        pltpu.VMEM((3, 16), jnp.float32),
        pltpu.VMEM_SHARED((_N, 8), jnp.float32),
        pltpu.SemaphoreType.DMA,
        pltpu.SemaphoreType.DMA,
    ])(_edge1_body)


def _node_body(acc_ref, c2_ref, u_ref, er_ref, m2_ref, smax):
    i = pl.program_id(0)
    a = acc_ref[0] + acc_ref[1]
    p1 = a[:, 0:4]
    q1 = a[:, 4:8]
    s1 = q1 / (p1 + 1e-9)
    ml = c2_ref[0, 0:4]
    mr = c2_ref[0, 4:8]
    el2 = jnp.sum(s1 * ml[None], axis=1) + c2_ref[0, 8]
    er2 = jnp.sum(s1 * mr[None], axis=1) + c2_ref[0, 9]
    u_ref[...] = jnp.concatenate(
        [el2[:, None], s1, jnp.zeros((_BN, 3), jnp.float32)], axis=1)
    er_ref[...] = er2[:, None]

    bl = jnp.max(el2)
    br = jnp.max(er2)

    @pl.when(i == 0)
    def _init():
        smax[0] = bl
        smax[1] = br

    @pl.when(i > 0)
    def _acc():
        smax[0] = jnp.maximum(smax[0], bl)
        smax[1] = jnp.maximum(smax[1], br)

    m2 = _leaky(smax[0] + smax[1])
    m2_ref[...] = jnp.full((1, 16), m2, jnp.float32)


def _run_node(acc1, c2):
    return pl.pallas_call(
        _node_body,
        grid=(_NBLK,),
        in_specs=[
            pl.BlockSpec((_NC, _BN, 8), lambda i: (0, i, 0)),
            pl.BlockSpec((8, 16), lambda i: (0, 0)),
        ],
        out_specs=[
            pl.BlockSpec((_BN, 8), lambda i: (i, 0)),
            pl.BlockSpec((_BN, 1), lambda i: (i, 0)),
            pl.BlockSpec((1, 16), lambda i: (0, 0)),
        ],
        out_shape=(jax.ShapeDtypeStruct((_N, 8), jnp.float32),
                   jax.ShapeDtypeStruct((_N, 1), jnp.float32),
                   jax.ShapeDtypeStruct((1, 16), jnp.float32)),
        scratch_shapes=[pltpu.SMEM((2,), jnp.float32)],
    )(acc1, c2)


def _edge2_body(src_hbm, dst_hbm, u_hbm, er_hbm, m2_hbm, zeros_hbm, out_hbm,
                srcv, dstv, urows, erd, rowsv, cv, acc, sem1, sem2):
    cid = lax.axis_index("c")
    sid = lax.axis_index("s")
    wid = sid * _NC + cid

    pltpu.sync_copy(m2_hbm, cv)

    @pl.when(sid == 0)
    def _zero():
        pltpu.sync_copy(zeros_hbm, acc)

    plsc.subcore_barrier()

    m2v = cv[0, :]
    lane = lax.iota(jnp.int32, 16)
    egrp = lane >> 3
    col = lane & 7
    zero16 = jnp.zeros((16,), jnp.int32)
    gcol = jnp.where(col < 5, col, zero16)

    def chunk(j, carry):
        base = wid * _EPW + j * _CH
        pltpu.sync_copy(src_hbm.at[pl.ds(base, _CH)], srcv)
        pltpu.sync_copy(dst_hbm.at[pl.ds(base, _CH)], dstv)
        cp1 = pltpu.async_copy(u_hbm.at[srcv], urows, sem1)
        cp2 = pltpu.async_copy(er_hbm.at[dstv], erd, sem2)
        cp1.wait()
        cp2.wait()

        def pair(p, c2):
            pidx = p * 2 + egrp
            el2s = plsc.load_gather(urows, [pidx, zero16])
            erdg = plsc.load_gather(erd, [pidx])
            pre = el2s + erdg
            p2 = jnp.exp(jnp.maximum(pre, _ALPHA * pre) - m2v)
            vals = plsc.load_gather(urows, [pidx, gcol])
            fac = jnp.where(col == 0, 1.0, jnp.where(col < 5, vals, 0.0))
            plsc.store_scatter(rowsv, [pidx, col], p2 * fac)
            return c2

        lax.fori_loop(0, _CH // 2, pair, 0, unroll=8)
        pltpu.sync_copy(rowsv, acc.at[dstv], add=True)
        return carry

    lax.fori_loop(0, _NCHUNK, chunk, 0)
    plsc.subcore_barrier()

    @pl.when(sid < 15)
    def _copy_out():
        r0 = sid * _RPT
        pltpu.sync_copy(acc.at[pl.ds(r0, _RPT)],
                        out_hbm.at[cid, pl.ds(r0, _RPT)])

    @pl.when(sid == 15)
    def _copy_last():
        r0 = 15 * _RPT
        pltpu.sync_copy(acc.at[pl.ds(r0, _RLAST)],
                        out_hbm.at[cid, pl.ds(r0, _RLAST)])


_edge2 = functools.partial(
    pl.kernel,
    out_type=jax.ShapeDtypeStruct((_NC, _N, 8), jnp.float32),
    mesh=plsc.VectorSubcoreMesh(core_axis_name="c", subcore_axis_name="s",
                                num_cores=_NC, num_subcores=_NS),
    compiler_params=pltpu.CompilerParams(
        use_tc_tiling_on_sc=False, needs_layout_passes=False),
    scratch_types=[
        pltpu.VMEM((_CH,), jnp.int32),
        pltpu.VMEM((_CH,), jnp.int32),
        pltpu.VMEM((_CH, 8), jnp.float32),
        pltpu.VMEM((_CH,), jnp.float32),
        pltpu.VMEM((_CH, 8), jnp.float32),
        pltpu.VMEM((1, 16), jnp.float32),
        pltpu.VMEM_SHARED((_N, 8), jnp.float32),
        pltpu.SemaphoreType.DMA,
        pltpu.SemaphoreType.DMA,
    ])(_edge2_body)


def _final_body(acc_ref, c2_ref, out_ref):
    a = acc_ref[0] + acc_ref[1]
    p2 = a[:, 0]
    t2 = a[:, 1:5]
    den = p2 + 1e-9
    m = c2_ref[1:5, :]
    c0 = c2_ref[5, :]
    b_out = c2_ref[6, :]
    v = (t2 / den[:, None]) @ m + (p2 / den)[:, None] * c0[None] + b_out[None]
    out_ref[...] = jnp.where(v > 0, v, jnp.exp(jnp.minimum(v, 0.0)) - 1.0)


def _run_final(acc2, c2):
    return pl.pallas_call(
        _final_body,
        grid=(_NBLK,),
        in_specs=[
            pl.BlockSpec((_NC, _BN, 8), lambda i: (0, i, 0)),
            pl.BlockSpec((8, 16), lambda i: (0, 0)),
        ],
        out_specs=pl.BlockSpec((_BN, 16), lambda i: (i, 0)),
        out_shape=jax.ShapeDtypeStruct((_N, 16), jnp.float32),
    )(acc2, c2)


@jax.jit
def kernel(X_in1, edge_index, metadata_in1, W1, b1, W2, b2):
    x = X_in1.reshape(_N, 1)
    src = edge_index[0]
    dst = edge_index[1]
    c1, c2 = _run_hyper(x, metadata_in1, W1, b1, W2, b2)
    zeros = jnp.zeros((_N, 8), jnp.float32)
    acc1 = _edge1(src, dst, x.reshape(_N), c1, zeros)
    u, er2, m2 = _run_node(acc1, c2)
    acc2 = _edge2(src, dst, u, er2.reshape(_N), m2, zeros)
    return _run_final(acc2, c2)


# speedup vs baseline: 212.1104x; 1.5620x over previous
import functools
import jax
import jax.numpy as jnp
from jax import lax
from jax.experimental import pallas as pl
from jax.experimental.pallas import tpu as pltpu
from jax.experimental.pallas import tpu_sc as plsc

_N = 50000
_E = 800000
_H = 4
_D = 16
_ALPHA = 0.2
_NC = 2
_NS = 16
_NW = _NC * _NS
_EPW = _E // _NW
_CH = 1000
_NCHUNK = _EPW // _CH
_NG = _CH // 16
_BN = 5000
_NBLK = _N // _BN
_RPT = 3128
_RLAST = _N - 15 * _RPT


def _leaky(v):
    return jnp.maximum(v, _ALPHA * v)


def _hyper_body(x_ref, md_ref, w1_ref, b1_ref,
                wvw_ref, wvb_ref, bfw_ref, bfb_ref, alw_ref, alb_ref,
                arw_ref, arb_ref, w3w_ref, w3b_ref, bow_ref, bob_ref,
                aolw_ref, aolb_ref, aorw_ref, aorb_ref,
                g4_ref, g64_ref, g16_ref, rrep_ref, g4c_ref, ones_ref,
                out_ref):
    h = jnp.tanh(md_ref[...] @ w1_ref[...] + b1_ref[...])
    wvf = h @ wvw_ref[...] + wvb_ref[...]
    bfcf = h @ bfw_ref[...] + bfb_ref[...]
    alf = h @ alw_ref[...] + alb_ref[...]
    arf = h @ arw_ref[...] + arb_ref[...]
    w3f = h @ w3w_ref[...] + w3b_ref[...]
    b_out = h @ bow_ref[...] + bob_ref[...]
    a_l = h @ aolw_ref[...] + aolb_ref[...]
    a_r = h @ aorw_ref[...] + aorb_ref[...]

    g4 = g4_ref[...]
    cl = (wvf * alf) @ g4
    cr = (wvf * arf) @ g4
    wvt = jnp.concatenate([wvf] * 16, axis=1)
    bft = jnp.concatenate([bfcf] * 16, axis=1)
    m_flat = (w3f * wvt) @ g64_ref[...]
    c0 = (w3f * bft) @ g16_ref[...]
    alr = a_l @ rrep_ref[...]
    arr = a_r @ rrep_ref[...]
    g4c = g4c_ref[...]
    ml = (m_flat * alr) @ g4c
    mr = (m_flat * arr) @ g4c
    ones16 = ones_ref[...]
    cl2 = (c0 * a_l) @ ones16
    cr2 = (c0 * a_r) @ ones16

    xv = x_ref[...]
    mx = jnp.max(xv)
    mn = jnp.min(xv)
    ub = jnp.maximum(mx * cl, mn * cl) + jnp.maximum(mx * cr, mn * cr)
    m1 = _leaky(ub)

    z = jnp.zeros((1, 4), jnp.float32)
    out_ref[...] = jnp.concatenate(
        [cl, cr, m1, z, ml, mr, cl2, cr2, z[:, 0:2],
         m_flat, c0, b_out, z], axis=1)


def _run_hyper(x, metadata, w1, b1, w2, b2):
    hd = _H * _D
    o1, o2, o3 = 4 * hd, 4 * hd + _D * hd, 4 * hd + _D * hd + _D
    pieces = [(0, hd), (hd, 2 * hd), (2 * hd, 3 * hd), (3 * hd, 4 * hd),
              (o1, o2), (o2, o3), (o3, o3 + _D), (o3 + _D, o3 + 2 * _D)]
    args = [x, metadata, w1, b1[None]]
    for lo, hi in pieces:
        args.append(w2[:, lo:hi])
        args.append(b2[lo:hi][None])
    i64 = jnp.arange(64)
    i1024 = jnp.arange(1024)
    i16 = jnp.arange(16)
    g4 = (i64[:, None] // 16 == jnp.arange(4)[None]).astype(jnp.float32)
    kh = (i1024 // 64) * 4 + (i1024 % 64) // 16
    g64 = (kh[:, None] == i64[None]).astype(jnp.float32)
    g16 = (i1024[:, None] // 64 == i16[None]).astype(jnp.float32)
    rrep = (i16[:, None] == i64[None] // 4).astype(jnp.float32)
    g4c = (i64[:, None] % 4 == jnp.arange(4)[None]).astype(jnp.float32)
    ones16 = jnp.ones((16, 1), jnp.float32)
    args += [g4, g64, g16, rrep, g4c, ones16]
    cf = pl.pallas_call(
        _hyper_body,
        out_shape=jax.ShapeDtypeStruct((1, 128), jnp.float32),
    )(*args)[0]
    c1 = jnp.repeat(cf[0:12], 16).reshape(12, 16)
    m = cf[28:92].reshape(16, 4).T
    row0 = jnp.concatenate([cf[16:24], cf[24:26], jnp.zeros((6,), jnp.float32)])
    c2 = jnp.concatenate([row0[None], m, cf[92:108][None], cf[108:124][None],
                          jnp.zeros((1, 16), jnp.float32)], axis=0)
    return c1, c2


def _edge1_body(src_hbm, dst_hbm, x_hbm, c1_hbm, zeros_hbm, out_hbm,
                srcv, dstv, xsv, xdv, rowsv, cv, acc, sem1, sem2):
    cid = lax.axis_index("c")
    sid = lax.axis_index("s")
    wid = sid * _NC + cid

    pltpu.sync_copy(c1_hbm, cv)

    @pl.when(sid == 0)
    def _zero():
        pltpu.sync_copy(zeros_hbm, acc)

    plsc.subcore_barrier()

    lane = lax.iota(jnp.int32, 16)
    tail_mask = lane < (_CH - _NG * 16)
    consts = [(cv[h, :], cv[4 + h, :], cv[8 + h, :]) for h in range(_H)]
    hcols = [jnp.full((16,), h, jnp.int32) for h in range(2 * _H)]

    def group(g, mask):
        row16 = g * 16 + lane
        xs = xsv[pl.ds(g * 16, 16)]
        xd = xdv[pl.ds(g * 16, 16)]
        for h in range(_H):
            clh, crh, m1h = consts[h]
            pre = xs * clh + xd * crh
            pz = jnp.exp(jnp.maximum(pre, _ALPHA * pre) - m1h)
            plsc.store_scatter(rowsv, [row16, hcols[h]], pz, mask=mask)
            plsc.store_scatter(rowsv, [row16, hcols[_H + h]], pz * xs,
                               mask=mask)

    def chunk(j, carry):
        base = wid * _EPW + j * _CH
        pltpu.sync_copy(src_hbm.at[pl.ds(base, _CH)], srcv)
        pltpu.sync_copy(dst_hbm.at[pl.ds(base, _CH)], dstv)
        cp1 = pltpu.async_copy(x_hbm.at[srcv], xsv.at[pl.ds(0, _CH)], sem1)
        cp2 = pltpu.async_copy(x_hbm.at[dstv], xdv.at[pl.ds(0, _CH)], sem2)
        cp1.wait()
        cp2.wait()

        def body(g, c2):
            group(g, None)
            return c2

        lax.fori_loop(0, _NG, body, 0, unroll=4)
        group(_NG, tail_mask)
        pltpu.sync_copy(rowsv.at[pl.ds(0, _CH)], acc.at[dstv], add=True)
        return carry

    lax.fori_loop(0, _NCHUNK, chunk, 0)
    plsc.subcore_barrier()

    @pl.when(sid < 15)
    def _copy_out():
        r0 = sid * _RPT
        pltpu.sync_copy(acc.at[pl.ds(r0, _RPT)],
                        out_hbm.at[cid, pl.ds(r0, _RPT)])

    @pl.when(sid == 15)
    def _copy_last():
        r0 = 15 * _RPT
        pltpu.sync_copy(acc.at[pl.ds(r0, _RLAST)],
                        out_hbm.at[cid, pl.ds(r0, _RLAST)])


_edge1 = functools.partial(
    pl.kernel,
    out_type=jax.ShapeDtypeStruct((_NC, _N, 8), jnp.float32),
    mesh=plsc.VectorSubcoreMesh(core_axis_name="c", subcore_axis_name="s",
                                num_cores=_NC, num_subcores=_NS),
    compiler_params=pltpu.CompilerParams(
        use_tc_tiling_on_sc=False, needs_layout_passes=False),
    scratch_types=[
        pltpu.VMEM((_CH,), jnp.int32),
        pltpu.VMEM((_CH,), jnp.int32),
        pltpu.VMEM((_CH + 16,), jnp.float32),
        pltpu.VMEM((_CH + 16,), jnp.float32),
        pltpu.VMEM((_CH + 16, 8), jnp.float32),
        pltpu.VMEM((12, 16), jnp.float32),
        pltpu.VMEM_SHARED((_N, 8), jnp.float32),
        pltpu.SemaphoreType.DMA,
        pltpu.SemaphoreType.DMA,
    ])(_edge1_body)


def _node_body(acc_ref, c2_ref, u_ref, er_ref, m2_ref, smax):
    i = pl.program_id(0)
    a = acc_ref[0] + acc_ref[1]
    p1 = a[:, 0:4]
    q1 = a[:, 4:8]
    s1 = q1 / (p1 + 1e-9)
    ml = c2_ref[0, 0:4]
    mr = c2_ref[0, 4:8]
    el2 = jnp.sum(s1 * ml[None], axis=1) + c2_ref[0, 8]
    er2 = jnp.sum(s1 * mr[None], axis=1) + c2_ref[0, 9]
    u_ref[...] = jnp.concatenate(
        [el2[:, None], s1, jnp.zeros((_BN, 3), jnp.float32)], axis=1)
    er_ref[...] = er2[:, None]

    bl = jnp.max(el2)
    br = jnp.max(er2)

    @pl.when(i == 0)
    def _init():
        smax[0] = bl
        smax[1] = br

    @pl.when(i > 0)
    def _acc():
        smax[0] = jnp.maximum(smax[0], bl)
        smax[1] = jnp.maximum(smax[1], br)

    m2 = _leaky(smax[0] + smax[1])
    m2_ref[...] = jnp.full((1, 16), m2, jnp.float32)


def _run_node(acc1, c2):
    return pl.pallas_call(
        _node_body,
        grid=(_NBLK,),
        in_specs=[
            pl.BlockSpec((_NC, _BN, 8), lambda i: (0, i, 0)),
            pl.BlockSpec((8, 16), lambda i: (0, 0)),
        ],
        out_specs=[
            pl.BlockSpec((_BN, 8), lambda i: (i, 0)),
            pl.BlockSpec((_BN, 1), lambda i: (i, 0)),
            pl.BlockSpec((1, 16), lambda i: (0, 0)),
        ],
        out_shape=(jax.ShapeDtypeStruct((_N, 8), jnp.float32),
                   jax.ShapeDtypeStruct((_N, 1), jnp.float32),
                   jax.ShapeDtypeStruct((1, 16), jnp.float32)),
        scratch_shapes=[pltpu.SMEM((2,), jnp.float32)],
    )(acc1, c2)


def _edge2_body(src_hbm, dst_hbm, u_hbm, er_hbm, m2_hbm, zeros_hbm, out_hbm,
                srcv, dstv, urows, erd, rowsv, cv, acc, sem1, sem2):
    cid = lax.axis_index("c")
    sid = lax.axis_index("s")
    wid = sid * _NC + cid

    pltpu.sync_copy(m2_hbm, cv)

    @pl.when(sid == 0)
    def _zero():
        pltpu.sync_copy(zeros_hbm, acc)

    plsc.subcore_barrier()

    m2v = cv[0, :]
    lane = lax.iota(jnp.int32, 16)
    tail_mask = lane < (_CH - _NG * 16)
    zero16 = jnp.zeros((16,), jnp.int32)
    hcols = [jnp.full((16,), h, jnp.int32) for h in range(5)]

    def zrow(r, c2):
        ridx = r * 2 + (lane >> 3)
        ccol = 5 + (lane & 7)
        zmask = (lane & 7) < 3
        plsc.store_scatter(rowsv, [ridx, ccol], jnp.zeros((16,), jnp.float32),
                           mask=zmask)
        return c2

    lax.fori_loop(0, (_CH + 16) // 2, zrow, 0, unroll=4)

    def group(g, mask):
        row16 = g * 16 + lane
        el2s = plsc.load_gather(urows, [row16, zero16])
        erdg = erd[pl.ds(g * 16, 16)]
        pre = el2s + erdg
        p2 = jnp.exp(jnp.maximum(pre, _ALPHA * pre) - m2v)
        plsc.store_scatter(rowsv, [row16, hcols[0]], p2, mask=mask)
        for h in range(1, 5):
            vals = plsc.load_gather(urows, [row16, hcols[h]])
            plsc.store_scatter(rowsv, [row16, hcols[h]], p2 * vals, mask=mask)

    def chunk(j, carry):
        base = wid * _EPW + j * _CH
        pltpu.sync_copy(src_hbm.at[pl.ds(base, _CH)], srcv)
        pltpu.sync_copy(dst_hbm.at[pl.ds(base, _CH)], dstv)
        cp1 = pltpu.async_copy(u_hbm.at[srcv], urows.at[pl.ds(0, _CH)], sem1)
        cp2 = pltpu.async_copy(er_hbm.at[dstv], erd.at[pl.ds(0, _CH)], sem2)
        cp1.wait()
        cp2.wait()

        def body(g, c2):
            group(g, None)
            return c2

        lax.fori_loop(0, _NG, body, 0, unroll=4)
        group(_NG, tail_mask)
        pltpu.sync_copy(rowsv.at[pl.ds(0, _CH)], acc.at[dstv], add=True)
        return carry

    lax.fori_loop(0, _NCHUNK, chunk, 0)
    plsc.subcore_barrier()

    @pl.when(sid < 15)
    def _copy_out():
        r0 = sid * _RPT
        pltpu.sync_copy(acc.at[pl.ds(r0, _RPT)],
                        out_hbm.at[cid, pl.ds(r0, _RPT)])

    @pl.when(sid == 15)
    def _copy_last():
        r0 = 15 * _RPT
        pltpu.sync_copy(acc.at[pl.ds(r0, _RLAST)],
                        out_hbm.at[cid, pl.ds(r0, _RLAST)])


_edge2 = functools.partial(
    pl.kernel,
    out_type=jax.ShapeDtypeStruct((_NC, _N, 8), jnp.float32),
    mesh=plsc.VectorSubcoreMesh(core_axis_name="c", subcore_axis_name="s",
                                num_cores=_NC, num_subcores=_NS),
    compiler_params=pltpu.CompilerParams(
        use_tc_tiling_on_sc=False, needs_layout_passes=False),
    scratch_types=[
        pltpu.VMEM((_CH,), jnp.int32),
        pltpu.VMEM((_CH,), jnp.int32),
        pltpu.VMEM((_CH + 16, 8), jnp.float32),
        pltpu.VMEM((_CH + 16,), jnp.float32),
        pltpu.VMEM((_CH + 16, 8), jnp.float32),
        pltpu.VMEM((1, 16), jnp.float32),
        pltpu.VMEM_SHARED((_N, 8), jnp.float32),
        pltpu.SemaphoreType.DMA,
        pltpu.SemaphoreType.DMA,
    ])(_edge2_body)


def _final_body(acc_ref, c2_ref, out_ref):
    a = acc_ref[0] + acc_ref[1]
    p2 = a[:, 0]
    t2 = a[:, 1:5]
    den = p2 + 1e-9
    m = c2_ref[1:5, :]
    c0 = c2_ref[5, :]
    b_out = c2_ref[6, :]
    v = (t2 / den[:, None]) @ m + (p2 / den)[:, None] * c0[None] + b_out[None]
    out_ref[...] = jnp.where(v > 0, v, jnp.exp(jnp.minimum(v, 0.0)) - 1.0)


def _run_final(acc2, c2):
    return pl.pallas_call(
        _final_body,
        grid=(_NBLK,),
        in_specs=[
            pl.BlockSpec((_NC, _BN, 8), lambda i: (0, i, 0)),
            pl.BlockSpec((8, 16), lambda i: (0, 0)),
        ],
        out_specs=pl.BlockSpec((_BN, 16), lambda i: (i, 0)),
        out_shape=jax.ShapeDtypeStruct((_N, 16), jnp.float32),
    )(acc2, c2)


@jax.jit
def kernel(X_in1, edge_index, metadata_in1, W1, b1, W2, b2):
    x = X_in1.reshape(_N, 1)
    src = edge_index[0]
    dst = edge_index[1]
    c1, c2 = _run_hyper(x, metadata_in1, W1, b1, W2, b2)
    zeros = jnp.zeros((_N, 8), jnp.float32)
    acc1 = _edge1(src, dst, x.reshape(_N), c1, zeros)
    u, er2, m2 = _run_node(acc1, c2)
    acc2 = _edge2(src, dst, u, er2.reshape(_N), m2, zeros)
    return _run_final(acc2, c2)


# speedup vs baseline: 272.1378x; 1.2830x over previous
import functools
import jax
import jax.numpy as jnp
from jax import lax
from jax.experimental import pallas as pl
from jax.experimental.pallas import tpu as pltpu
from jax.experimental.pallas import tpu_sc as plsc

_N = 50000
_E = 800000
_H = 4
_D = 16
_ALPHA = 0.2
_NC = 2
_NS = 16
_NW = _NC * _NS
_EPW = _E // _NW
_CH = 1000
_NCHUNK = _EPW // _CH
_NG = _CH // 16
_BN = 5000
_NBLK = _N // _BN
_RPT = 3128
_RLAST = _N - 15 * _RPT


def _leaky(v):
    return jnp.maximum(v, _ALPHA * v)


def _hyper_body(x_ref, md_ref, w1_ref, b1_ref,
                wvw_ref, wvb_ref, bfw_ref, bfb_ref, alw_ref, alb_ref,
                arw_ref, arb_ref, w3w_ref, w3b_ref, bow_ref, bob_ref,
                aolw_ref, aolb_ref, aorw_ref, aorb_ref,
                g4_ref, g64_ref, g16_ref, rrep_ref, g4c_ref, ones_ref,
                out_ref):
    h = jnp.tanh(md_ref[...] @ w1_ref[...] + b1_ref[...])
    wvf = h @ wvw_ref[...] + wvb_ref[...]
    bfcf = h @ bfw_ref[...] + bfb_ref[...]
    alf = h @ alw_ref[...] + alb_ref[...]
    arf = h @ arw_ref[...] + arb_ref[...]
    w3f = h @ w3w_ref[...] + w3b_ref[...]
    b_out = h @ bow_ref[...] + bob_ref[...]
    a_l = h @ aolw_ref[...] + aolb_ref[...]
    a_r = h @ aorw_ref[...] + aorb_ref[...]

    g4 = g4_ref[...]
    cl = (wvf * alf) @ g4
    cr = (wvf * arf) @ g4
    wvt = jnp.concatenate([wvf] * 16, axis=1)
    bft = jnp.concatenate([bfcf] * 16, axis=1)
    m_flat = (w3f * wvt) @ g64_ref[...]
    c0 = (w3f * bft) @ g16_ref[...]
    alr = a_l @ rrep_ref[...]
    arr = a_r @ rrep_ref[...]
    g4c = g4c_ref[...]
    ml = (m_flat * alr) @ g4c
    mr = (m_flat * arr) @ g4c
    ones16 = ones_ref[...]
    cl2 = (c0 * a_l) @ ones16
    cr2 = (c0 * a_r) @ ones16

    xv = x_ref[...]
    mx = jnp.max(xv)
    mn = jnp.min(xv)
    ub = jnp.maximum(mx * cl, mn * cl) + jnp.maximum(mx * cr, mn * cr)
    m1 = _leaky(ub)

    maxax = jnp.maximum(mx, -mn)
    ub2 = (maxax * (jnp.sum(jnp.abs(ml)) + jnp.sum(jnp.abs(mr)))
           + cl2[0, 0] + cr2[0, 0])
    m2b = jnp.full((1, 1), _leaky(ub2), jnp.float32)

    z = jnp.zeros((1, 4), jnp.float32)
    out_ref[...] = jnp.concatenate(
        [cl, cr, m1, z, ml, mr, cl2, cr2, m2b, z[:, 0:1],
         m_flat, c0, b_out, z], axis=1)


def _run_hyper(x, metadata, w1, b1, w2, b2):
    hd = _H * _D
    o1, o2, o3 = 4 * hd, 4 * hd + _D * hd, 4 * hd + _D * hd + _D
    pieces = [(0, hd), (hd, 2 * hd), (2 * hd, 3 * hd), (3 * hd, 4 * hd),
              (o1, o2), (o2, o3), (o3, o3 + _D), (o3 + _D, o3 + 2 * _D)]
    args = [x, metadata, w1, b1[None]]
    for lo, hi in pieces:
        args.append(w2[:, lo:hi])
        args.append(b2[lo:hi][None])
    i64 = jnp.arange(64)
    i1024 = jnp.arange(1024)
    i16 = jnp.arange(16)
    g4 = (i64[:, None] // 16 == jnp.arange(4)[None]).astype(jnp.float32)
    kh = (i1024 // 64) * 4 + (i1024 % 64) // 16
    g64 = (kh[:, None] == i64[None]).astype(jnp.float32)
    g16 = (i1024[:, None] // 64 == i16[None]).astype(jnp.float32)
    rrep = (i16[:, None] == i64[None] // 4).astype(jnp.float32)
    g4c = (i64[:, None] % 4 == jnp.arange(4)[None]).astype(jnp.float32)
    ones16 = jnp.ones((16, 1), jnp.float32)
    args += [g4, g64, g16, rrep, g4c, ones16]
    cf = pl.pallas_call(
        _hyper_body,
        out_shape=jax.ShapeDtypeStruct((1, 128), jnp.float32),
    )(*args)[0]
    c1 = jnp.repeat(cf[0:12], 16).reshape(12, 16)
    c3 = jnp.repeat(cf[16:27], 16).reshape(11, 16)
    m = cf[28:92].reshape(16, 4).T
    row0 = jnp.concatenate([cf[16:24], cf[24:26], jnp.zeros((6,), jnp.float32)])
    c2 = jnp.concatenate([row0[None], m, cf[92:108][None], cf[108:124][None],
                          jnp.zeros((1, 16), jnp.float32)], axis=0)
    return c1, c2, c3


def _edge1_body(src_hbm, dst_hbm, x_hbm, c1_hbm, zeros_hbm, out_hbm,
                srcv, dstv, xsv, xdv, rowsv, cv, acc, sem1, sem2):
    cid = lax.axis_index("c")
    sid = lax.axis_index("s")
    wid = sid * _NC + cid

    pltpu.sync_copy(c1_hbm, cv)

    @pl.when(sid == 0)
    def _zero():
        pltpu.sync_copy(zeros_hbm, acc)

    plsc.subcore_barrier()

    lane = lax.iota(jnp.int32, 16)
    tail_mask = lane < (_CH - _NG * 16)
    consts = [(cv[h, :], cv[4 + h, :], cv[8 + h, :]) for h in range(_H)]
    hcols = [jnp.full((16,), h, jnp.int32) for h in range(2 * _H)]

    def group(g, mask):
        row16 = g * 16 + lane
        xs = xsv[pl.ds(g * 16, 16)]
        xd = xdv[pl.ds(g * 16, 16)]
        for h in range(_H):
            clh, crh, m1h = consts[h]
            pre = xs * clh + xd * crh
            pz = jnp.exp(jnp.maximum(pre, _ALPHA * pre) - m1h)
            plsc.store_scatter(rowsv, [row16, hcols[h]], pz, mask=mask)
            plsc.store_scatter(rowsv, [row16, hcols[_H + h]], pz * xs,
                               mask=mask)

    def chunk(j, carry):
        base = wid * _EPW + j * _CH
        pltpu.sync_copy(src_hbm.at[pl.ds(base, _CH)], srcv)
        pltpu.sync_copy(dst_hbm.at[pl.ds(base, _CH)], dstv)
        cp1 = pltpu.async_copy(x_hbm.at[srcv], xsv.at[pl.ds(0, _CH)], sem1)
        cp2 = pltpu.async_copy(x_hbm.at[dstv], xdv.at[pl.ds(0, _CH)], sem2)
        cp1.wait()
        cp2.wait()

        def body(g, c2):
            group(g, None)
            return c2

        lax.fori_loop(0, _NG, body, 0, unroll=4)
        group(_NG, tail_mask)
        pltpu.sync_copy(rowsv.at[pl.ds(0, _CH)], acc.at[dstv], add=True)
        return carry

    lax.fori_loop(0, _NCHUNK, chunk, 0)
    plsc.subcore_barrier()

    @pl.when(sid < 15)
    def _copy_out():
        r0 = sid * _RPT
        pltpu.sync_copy(acc.at[pl.ds(r0, _RPT)],
                        out_hbm.at[cid, pl.ds(r0, _RPT)])

    @pl.when(sid == 15)
    def _copy_last():
        r0 = 15 * _RPT
        pltpu.sync_copy(acc.at[pl.ds(r0, _RLAST)],
                        out_hbm.at[cid, pl.ds(r0, _RLAST)])


_edge1 = functools.partial(
    pl.kernel,
    out_type=jax.ShapeDtypeStruct((_NC, _N, 8), jnp.float32),
    mesh=plsc.VectorSubcoreMesh(core_axis_name="c", subcore_axis_name="s",
                                num_cores=_NC, num_subcores=_NS),
    compiler_params=pltpu.CompilerParams(
        use_tc_tiling_on_sc=False, needs_layout_passes=False),
    scratch_types=[
        pltpu.VMEM((_CH,), jnp.int32),
        pltpu.VMEM((_CH,), jnp.int32),
        pltpu.VMEM((_CH + 16,), jnp.float32),
        pltpu.VMEM((_CH + 16,), jnp.float32),
        pltpu.VMEM((_CH + 16, 8), jnp.float32),
        pltpu.VMEM((12, 16), jnp.float32),
        pltpu.VMEM_SHARED((_N, 8), jnp.float32),
        pltpu.SemaphoreType.DMA,
        pltpu.SemaphoreType.DMA,
    ])(_edge1_body)


_SZ = 512
_NSUB = _RPT // _SZ


def _edge2_body(src_hbm, dst_hbm, acc1_hbm, c3_hbm, zeros_hbm, out_hbm,
                srcv, dstv, urows, erd, rowsv, cv, a0v, a1v, ubuf, erbuf,
                ush, ersh, acc, sem1, sem2):
    cid = lax.axis_index("c")
    sid = lax.axis_index("s")
    wid = sid * _NC + cid

    pltpu.sync_copy(c3_hbm, cv)

    @pl.when(sid == 0)
    def _zero():
        pltpu.sync_copy(zeros_hbm, acc)

    lane = lax.iota(jnp.int32, 16)
    zero16 = jnp.zeros((16,), jnp.int32)
    hcols = [jnp.full((16,), h, jnp.int32) for h in range(8)]
    mlv = [cv[h, :] for h in range(_H)]
    mrv = [cv[_H + h, :] for h in range(_H)]
    cl2v = cv[8, :]
    cr2v = cv[9, :]
    m2v = cv[10, :]

    def p1_group(g, r0, mask):
        row16 = g * 16 + lane
        a0c = [plsc.load_gather(a0v, [row16, hcols[c]]) for c in range(8)]
        a1c = [plsc.load_gather(a1v, [row16, hcols[c]]) for c in range(8)]
        s1 = [(a0c[4 + h] + a1c[4 + h]) / (a0c[h] + a1c[h] + 1e-9)
              for h in range(_H)]
        el2 = s1[0] * mlv[0] + s1[1] * mlv[1] + s1[2] * mlv[2] \
            + s1[3] * mlv[3] + cl2v
        er2 = s1[0] * mrv[0] + s1[1] * mrv[1] + s1[2] * mrv[2] \
            + s1[3] * mrv[3] + cr2v
        plsc.store_scatter(ubuf, [row16, hcols[0]], el2, mask=mask)
        for h in range(_H):
            plsc.store_scatter(ubuf, [row16, hcols[1 + h]], s1[h], mask=mask)
        plsc.store_scatter(erbuf, [row16], er2, mask=mask)

    def p1_block(r0, sz):
        pltpu.sync_copy(acc1_hbm.at[0, pl.ds(r0, sz)], a0v.at[pl.ds(0, sz)])
        pltpu.sync_copy(acc1_hbm.at[1, pl.ds(r0, sz)], a1v.at[pl.ds(0, sz)])

        def body(g, c):
            p1_group(g, r0, None)
            return c

        lax.fori_loop(0, sz // 16, body, 0, unroll=2)
        if sz % 16:
            p1_group(sz // 16, r0, lane < (sz % 16))
        pltpu.sync_copy(ubuf.at[pl.ds(0, sz)], ush.at[pl.ds(r0, sz)])
        pltpu.sync_copy(erbuf.at[pl.ds(0, sz)], ersh.at[pl.ds(r0, sz)])

    r0 = sid * _RPT

    def p1_sub(k, c):
        p1_block(r0 + k * _SZ, _SZ)
        return c

    lax.fori_loop(0, _NSUB, p1_sub, 0)

    @pl.when(sid < 15)
    def _p1_rem():
        p1_block(r0 + _NSUB * _SZ, _RPT - _NSUB * _SZ)

    @pl.when(sid == 15)
    def _p1_rem_last():
        p1_block(r0 + _NSUB * _SZ, _RLAST - _NSUB * _SZ)

    plsc.subcore_barrier()

    tail_mask = lane < (_CH - _NG * 16)

    def zrow(r, c2):
        ridx = r * 2 + (lane >> 3)
        ccol = 5 + (lane & 7)
        zmask = (lane & 7) < 3
        plsc.store_scatter(rowsv, [ridx, ccol], jnp.zeros((16,), jnp.float32),
                           mask=zmask)
        return c2

    lax.fori_loop(0, (_CH + 16) // 2, zrow, 0, unroll=4)

    def group(g, mask):
        row16 = g * 16 + lane
        el2s = plsc.load_gather(urows, [row16, zero16])
        erdg = erd[pl.ds(g * 16, 16)]
        pre = el2s + erdg
        p2 = jnp.exp(jnp.maximum(pre, _ALPHA * pre) - m2v)
        plsc.store_scatter(rowsv, [row16, hcols[0]], p2, mask=mask)
        for h in range(1, 5):
            vals = plsc.load_gather(urows, [row16, hcols[h]])
            plsc.store_scatter(rowsv, [row16, hcols[h]], p2 * vals, mask=mask)

    def chunk(j, carry):
        base = wid * _EPW + j * _CH
        pltpu.sync_copy(src_hbm.at[pl.ds(base, _CH)], srcv)
        pltpu.sync_copy(dst_hbm.at[pl.ds(base, _CH)], dstv)
        cp1 = pltpu.async_copy(ush.at[srcv], urows.at[pl.ds(0, _CH)], sem1)
        cp2 = pltpu.async_copy(ersh.at[dstv], erd.at[pl.ds(0, _CH)], sem2)
        cp1.wait()
        cp2.wait()

        def body(g, c2):
            group(g, None)
            return c2

        lax.fori_loop(0, _NG, body, 0, unroll=4)
        group(_NG, tail_mask)
        pltpu.sync_copy(rowsv.at[pl.ds(0, _CH)], acc.at[dstv], add=True)
        return carry

    lax.fori_loop(0, _NCHUNK, chunk, 0)
    plsc.subcore_barrier()

    @pl.when(sid < 15)
    def _copy_out():
        r0 = sid * _RPT
        pltpu.sync_copy(acc.at[pl.ds(r0, _RPT)],
                        out_hbm.at[cid, pl.ds(r0, _RPT)])

    @pl.when(sid == 15)
    def _copy_last():
        r0 = 15 * _RPT
        pltpu.sync_copy(acc.at[pl.ds(r0, _RLAST)],
                        out_hbm.at[cid, pl.ds(r0, _RLAST)])


_edge2 = functools.partial(
    pl.kernel,
    out_type=jax.ShapeDtypeStruct((_NC, _N, 8), jnp.float32),
    mesh=plsc.VectorSubcoreMesh(core_axis_name="c", subcore_axis_name="s",
                                num_cores=_NC, num_subcores=_NS),
    compiler_params=pltpu.CompilerParams(
        use_tc_tiling_on_sc=False, needs_layout_passes=False),
    scratch_types=[
        pltpu.VMEM((_CH,), jnp.int32),
        pltpu.VMEM((_CH,), jnp.int32),
        pltpu.VMEM((_CH + 16, 8), jnp.float32),
        pltpu.VMEM((_CH + 16,), jnp.float32),
        pltpu.VMEM((_CH + 16, 8), jnp.float32),
        pltpu.VMEM((11, 16), jnp.float32),
        pltpu.VMEM((_SZ, 8), jnp.float32),
        pltpu.VMEM((_SZ, 8), jnp.float32),
        pltpu.VMEM((_SZ, 8), jnp.float32),
        pltpu.VMEM((_SZ,), jnp.float32),
        pltpu.VMEM_SHARED((_N, 8), jnp.float32),
        pltpu.VMEM_SHARED((_N,), jnp.float32),
        pltpu.VMEM_SHARED((_N, 8), jnp.float32),
        pltpu.SemaphoreType.DMA,
        pltpu.SemaphoreType.DMA,
    ])(_edge2_body)


def _final_body(acc_ref, c2_ref, out_ref):
    a = acc_ref[0] + acc_ref[1]
    p2 = a[:, 0]
    t2 = a[:, 1:5]
    den = p2 + 1e-9
    m = c2_ref[1:5, :]
    c0 = c2_ref[5, :]
    b_out = c2_ref[6, :]
    v = (t2 / den[:, None]) @ m + (p2 / den)[:, None] * c0[None] + b_out[None]
    out_ref[...] = jnp.where(v > 0, v, jnp.exp(jnp.minimum(v, 0.0)) - 1.0)


def _run_final(acc2, c2):
    return pl.pallas_call(
        _final_body,
        grid=(_NBLK,),
        in_specs=[
            pl.BlockSpec((_NC, _BN, 8), lambda i: (0, i, 0)),
            pl.BlockSpec((8, 16), lambda i: (0, 0)),
        ],
        out_specs=pl.BlockSpec((_BN, 16), lambda i: (i, 0)),
        out_shape=jax.ShapeDtypeStruct((_N, 16), jnp.float32),
    )(acc2, c2)


@jax.jit
def kernel(X_in1, edge_index, metadata_in1, W1, b1, W2, b2):
    x = X_in1.reshape(_N, 1)
    src = edge_index[0]
    dst = edge_index[1]
    c1, c2, c3 = _run_hyper(x, metadata_in1, W1, b1, W2, b2)
    zeros = jnp.zeros((_N, 8), jnp.float32)
    acc1 = _edge1(src, dst, x.reshape(_N), c1, zeros)
    acc2 = _edge2(src, dst, acc1, c3, zeros)
    return _run_final(acc2, c2)


# speedup vs baseline: 277.9573x; 1.0214x over previous
import functools
import jax
import jax.numpy as jnp
from jax import lax
from jax.experimental import pallas as pl
from jax.experimental.pallas import tpu as pltpu
from jax.experimental.pallas import tpu_sc as plsc

_N = 50000
_E = 800000
_H = 4
_D = 16
_ALPHA = 0.2
_NC = 2
_NS = 16
_NW = _NC * _NS
_EPW = _E // _NW
_CH = 1000
_NCHUNK = _EPW // _CH
_NG = _CH // 16
_BN = 5000
_NBLK = _N // _BN
_RPT = 3128
_RLAST = _N - 15 * _RPT


def _leaky(v):
    return jnp.maximum(v, _ALPHA * v)


def _hyper_body(x_ref, md_ref, w1_ref, b1_ref,
                wvw_ref, wvb_ref, bfw_ref, bfb_ref, alw_ref, alb_ref,
                arw_ref, arb_ref, w3w_ref, w3b_ref, bow_ref, bob_ref,
                aolw_ref, aolb_ref, aorw_ref, aorb_ref,
                g4_ref, g64_ref, g16_ref, rrep_ref, g4c_ref, ones_ref,
                out_ref):
    h = jnp.tanh(md_ref[...] @ w1_ref[...] + b1_ref[...])
    wvf = h @ wvw_ref[...] + wvb_ref[...]
    bfcf = h @ bfw_ref[...] + bfb_ref[...]
    alf = h @ alw_ref[...] + alb_ref[...]
    arf = h @ arw_ref[...] + arb_ref[...]
    w3f = h @ w3w_ref[...] + w3b_ref[...]
    b_out = h @ bow_ref[...] + bob_ref[...]
    a_l = h @ aolw_ref[...] + aolb_ref[...]
    a_r = h @ aorw_ref[...] + aorb_ref[...]

    g4 = g4_ref[...]
    cl = (wvf * alf) @ g4
    cr = (wvf * arf) @ g4
    wvt = jnp.concatenate([wvf] * 16, axis=1)
    bft = jnp.concatenate([bfcf] * 16, axis=1)
    m_flat = (w3f * wvt) @ g64_ref[...]
    c0 = (w3f * bft) @ g16_ref[...]
    alr = a_l @ rrep_ref[...]
    arr = a_r @ rrep_ref[...]
    g4c = g4c_ref[...]
    ml = (m_flat * alr) @ g4c
    mr = (m_flat * arr) @ g4c
    ones16 = ones_ref[...]
    cl2 = (c0 * a_l) @ ones16
    cr2 = (c0 * a_r) @ ones16

    xv = x_ref[...]
    mx = jnp.max(xv)
    mn = jnp.min(xv)
    ub = jnp.maximum(mx * cl, mn * cl) + jnp.maximum(mx * cr, mn * cr)
    m1 = _leaky(ub)

    maxax = jnp.maximum(mx, -mn)
    ub2 = (maxax * (jnp.sum(jnp.abs(ml)) + jnp.sum(jnp.abs(mr)))
           + cl2[0, 0] + cr2[0, 0])
    m2b = jnp.full((1, 1), _leaky(ub2), jnp.float32)

    z = jnp.zeros((1, 4), jnp.float32)
    out_ref[...] = jnp.concatenate(
        [cl, cr, m1, z, ml, mr, cl2, cr2, m2b, z[:, 0:1],
         m_flat, c0, b_out, z], axis=1)


def _run_hyper(x, metadata, w1, b1, w2, b2):
    hd = _H * _D
    o1, o2, o3 = 4 * hd, 4 * hd + _D * hd, 4 * hd + _D * hd + _D
    pieces = [(0, hd), (hd, 2 * hd), (2 * hd, 3 * hd), (3 * hd, 4 * hd),
              (o1, o2), (o2, o3), (o3, o3 + _D), (o3 + _D, o3 + 2 * _D)]
    args = [x, metadata, w1, b1[None]]
    for lo, hi in pieces:
        args.append(w2[:, lo:hi])
        args.append(b2[lo:hi][None])
    i64 = jnp.arange(64)
    i1024 = jnp.arange(1024)
    i16 = jnp.arange(16)
    g4 = (i64[:, None] // 16 == jnp.arange(4)[None]).astype(jnp.float32)
    kh = (i1024 // 64) * 4 + (i1024 % 64) // 16
    g64 = (kh[:, None] == i64[None]).astype(jnp.float32)
    g16 = (i1024[:, None] // 64 == i16[None]).astype(jnp.float32)
    rrep = (i16[:, None] == i64[None] // 4).astype(jnp.float32)
    g4c = (i64[:, None] % 4 == jnp.arange(4)[None]).astype(jnp.float32)
    ones16 = jnp.ones((16, 1), jnp.float32)
    args += [g4, g64, g16, rrep, g4c, ones16]
    cf = pl.pallas_call(
        _hyper_body,
        out_shape=jax.ShapeDtypeStruct((1, 128), jnp.float32),
    )(*args)[0]
    c1 = jnp.repeat(cf[0:12], 16).reshape(12, 16)
    c3 = jnp.repeat(cf[16:27], 16).reshape(11, 16)
    m = cf[28:92].reshape(16, 4).T
    row0 = jnp.concatenate([cf[16:24], cf[24:26], jnp.zeros((6,), jnp.float32)])
    c2 = jnp.concatenate([row0[None], m, cf[92:108][None], cf[108:124][None],
                          jnp.zeros((1, 16), jnp.float32)], axis=0)
    return c1, c2, c3


def _edge1_body(src_hbm, dst_hbm, x_hbm, c1_hbm, zeros_hbm, out_hbm,
                srcv, dstv0, dstv1, xsv, xdv, rowsv0, rowsv1, cv, acc,
                sem1, sem2, sems0, sems1):
    cid = lax.axis_index("c")
    sid = lax.axis_index("s")
    wid = sid * _NC + cid

    pltpu.sync_copy(c1_hbm, cv)

    @pl.when(sid == 0)
    def _zero():
        pltpu.sync_copy(zeros_hbm, acc)

    plsc.subcore_barrier()

    lane = lax.iota(jnp.int32, 16)
    tail_mask = lane < (_CH - _NG * 16)
    consts = [(cv[h, :], cv[4 + h, :], cv[8 + h, :]) for h in range(_H)]
    hcols = [jnp.full((16,), h, jnp.int32) for h in range(2 * _H)]
    bufs = [(dstv0, rowsv0, sems0), (dstv1, rowsv1, sems1)]

    def group(g, rowsv, mask):
        row16 = g * 16 + lane
        xs = xsv[pl.ds(g * 16, 16)]
        xd = xdv[pl.ds(g * 16, 16)]
        for h in range(_H):
            clh, crh, m1h = consts[h]
            pre = xs * clh + xd * crh
            pz = jnp.exp(jnp.maximum(pre, _ALPHA * pre) - m1h)
            plsc.store_scatter(rowsv, [row16, hcols[h]], pz, mask=mask)
            plsc.store_scatter(rowsv, [row16, hcols[_H + h]], pz * xs,
                               mask=mask)

    def chunk(j, dstv, rowsv, sems):
        @pl.when(j >= 2)
        def _drain():
            pltpu.make_async_copy(
                rowsv.at[pl.ds(0, _CH)], acc.at[dstv], sems).wait()

        base = wid * _EPW + j * _CH
        pltpu.sync_copy(src_hbm.at[pl.ds(base, _CH)], srcv)
        pltpu.sync_copy(dst_hbm.at[pl.ds(base, _CH)], dstv)
        cp1 = pltpu.async_copy(x_hbm.at[srcv], xsv.at[pl.ds(0, _CH)], sem1)
        cp2 = pltpu.async_copy(x_hbm.at[dstv], xdv.at[pl.ds(0, _CH)], sem2)
        cp1.wait()
        cp2.wait()

        def body(g, c2):
            group(g, rowsv, None)
            return c2

        lax.fori_loop(0, _NG, body, 0, unroll=4)
        group(_NG, rowsv, tail_mask)
        pltpu.async_copy(rowsv.at[pl.ds(0, _CH)], acc.at[dstv], sems,
                         add=True)

    def dchunk(k, carry):
        j = k * 2
        chunk(j, *bufs[0])

        @pl.when(j + 1 < _NCHUNK)
        def _odd():
            chunk(j + 1, *bufs[1])

        return carry

    lax.fori_loop(0, (_NCHUNK + 1) // 2, dchunk, 0)
    for dstv, rowsv, sems in bufs:
        pltpu.make_async_copy(
            rowsv.at[pl.ds(0, _CH)], acc.at[dstv], sems).wait()
    plsc.subcore_barrier()

    @pl.when(sid < 15)
    def _copy_out():
        r0 = sid * _RPT
        pltpu.sync_copy(acc.at[pl.ds(r0, _RPT)],
                        out_hbm.at[cid, pl.ds(r0, _RPT)])

    @pl.when(sid == 15)
    def _copy_last():
        r0 = 15 * _RPT
        pltpu.sync_copy(acc.at[pl.ds(r0, _RLAST)],
                        out_hbm.at[cid, pl.ds(r0, _RLAST)])


_edge1 = functools.partial(
    pl.kernel,
    out_type=jax.ShapeDtypeStruct((_NC, _N, 8), jnp.float32),
    mesh=plsc.VectorSubcoreMesh(core_axis_name="c", subcore_axis_name="s",
                                num_cores=_NC, num_subcores=_NS),
    compiler_params=pltpu.CompilerParams(
        use_tc_tiling_on_sc=False, needs_layout_passes=False),
    scratch_types=[
        pltpu.VMEM((_CH,), jnp.int32),
        pltpu.VMEM((_CH,), jnp.int32),
        pltpu.VMEM((_CH,), jnp.int32),
        pltpu.VMEM((_CH + 16,), jnp.float32),
        pltpu.VMEM((_CH + 16,), jnp.float32),
        pltpu.VMEM((_CH + 16, 8), jnp.float32),
        pltpu.VMEM((_CH + 16, 8), jnp.float32),
        pltpu.VMEM((12, 16), jnp.float32),
        pltpu.VMEM_SHARED((_N, 8), jnp.float32),
        pltpu.SemaphoreType.DMA,
        pltpu.SemaphoreType.DMA,
        pltpu.SemaphoreType.DMA,
        pltpu.SemaphoreType.DMA,
    ])(_edge1_body)


_SZ = 256
_NSUB = _RPT // _SZ


def _edge2_body(src_hbm, dst_hbm, acc1_hbm, c3_hbm, zeros_hbm, out_hbm,
                srcv, dstv0, dstv1, urows, erd, rowsv0, rowsv1, cv,
                a0v, a1v, ubuf, erbuf, ush, ersh, acc,
                sem1, sem2, sems0, sems1):
    cid = lax.axis_index("c")
    sid = lax.axis_index("s")
    wid = sid * _NC + cid

    pltpu.sync_copy(c3_hbm, cv)

    @pl.when(sid == 0)
    def _zero():
        pltpu.sync_copy(zeros_hbm, acc)

    lane = lax.iota(jnp.int32, 16)
    zero16 = jnp.zeros((16,), jnp.int32)
    hcols = [jnp.full((16,), h, jnp.int32) for h in range(8)]
    mlv = [cv[h, :] for h in range(_H)]
    mrv = [cv[_H + h, :] for h in range(_H)]
    cl2v = cv[8, :]
    cr2v = cv[9, :]
    m2v = cv[10, :]

    def p1_group(g, r0, mask):
        row16 = g * 16 + lane
        a0c = [plsc.load_gather(a0v, [row16, hcols[c]]) for c in range(8)]
        a1c = [plsc.load_gather(a1v, [row16, hcols[c]]) for c in range(8)]
        s1 = [(a0c[4 + h] + a1c[4 + h]) / (a0c[h] + a1c[h] + 1e-9)
              for h in range(_H)]
        el2 = s1[0] * mlv[0] + s1[1] * mlv[1] + s1[2] * mlv[2] \
            + s1[3] * mlv[3] + cl2v
        er2 = s1[0] * mrv[0] + s1[1] * mrv[1] + s1[2] * mrv[2] \
            + s1[3] * mrv[3] + cr2v
        plsc.store_scatter(ubuf, [row16, hcols[0]], el2, mask=mask)
        for h in range(_H):
            plsc.store_scatter(ubuf, [row16, hcols[1 + h]], s1[h], mask=mask)
        plsc.store_scatter(erbuf, [row16], er2, mask=mask)

    def p1_block(r0, sz):
        pltpu.sync_copy(acc1_hbm.at[0, pl.ds(r0, sz)], a0v.at[pl.ds(0, sz)])
        pltpu.sync_copy(acc1_hbm.at[1, pl.ds(r0, sz)], a1v.at[pl.ds(0, sz)])

        def body(g, c):
            p1_group(g, r0, None)
            return c

        lax.fori_loop(0, sz // 16, body, 0, unroll=2)
        if sz % 16:
            p1_group(sz // 16, r0, lane < (sz % 16))
        pltpu.sync_copy(ubuf.at[pl.ds(0, sz)], ush.at[pl.ds(r0, sz)])
        pltpu.sync_copy(erbuf.at[pl.ds(0, sz)], ersh.at[pl.ds(r0, sz)])

    r0 = sid * _RPT

    def p1_sub(k, c):
        p1_block(r0 + k * _SZ, _SZ)
        return c

    lax.fori_loop(0, _NSUB, p1_sub, 0)

    @pl.when(sid < 15)
    def _p1_rem():
        p1_block(r0 + _NSUB * _SZ, _RPT - _NSUB * _SZ)

    @pl.when(sid == 15)
    def _p1_rem_last():
        p1_block(r0 + _NSUB * _SZ, _RLAST - _NSUB * _SZ)

    plsc.subcore_barrier()

    tail_mask = lane < (_CH - _NG * 16)
    bufs = [(dstv0, rowsv0, sems0), (dstv1, rowsv1, sems1)]

    def zrow(r, c2):
        ridx = r * 2 + (lane >> 3)
        ccol = 5 + (lane & 7)
        zmask = (lane & 7) < 3
        zv = jnp.zeros((16,), jnp.float32)
        plsc.store_scatter(rowsv0, [ridx, ccol], zv, mask=zmask)
        plsc.store_scatter(rowsv1, [ridx, ccol], zv, mask=zmask)
        return c2

    lax.fori_loop(0, (_CH + 16) // 2, zrow, 0, unroll=4)

    def group(g, rowsv, mask):
        row16 = g * 16 + lane
        el2s = plsc.load_gather(urows, [row16, zero16])
        erdg = erd[pl.ds(g * 16, 16)]
        pre = el2s + erdg
        p2 = jnp.exp(jnp.maximum(pre, _ALPHA * pre) - m2v)
        plsc.store_scatter(rowsv, [row16, hcols[0]], p2, mask=mask)
        for h in range(1, 5):
            vals = plsc.load_gather(urows, [row16, hcols[h]])
            plsc.store_scatter(rowsv, [row16, hcols[h]], p2 * vals, mask=mask)

    def chunk(j, dstv, rowsv, sems):
        @pl.when(j >= 2)
        def _drain():
            pltpu.make_async_copy(
                rowsv.at[pl.ds(0, _CH)], acc.at[dstv], sems).wait()

        base = wid * _EPW + j * _CH
        pltpu.sync_copy(src_hbm.at[pl.ds(base, _CH)], srcv)
        pltpu.sync_copy(dst_hbm.at[pl.ds(base, _CH)], dstv)
        cp1 = pltpu.async_copy(ush.at[srcv], urows.at[pl.ds(0, _CH)], sem1)
        cp2 = pltpu.async_copy(ersh.at[dstv], erd.at[pl.ds(0, _CH)], sem2)
        cp1.wait()
        cp2.wait()

        def body(g, c2):
            group(g, rowsv, None)
            return c2

        lax.fori_loop(0, _NG, body, 0, unroll=4)
        group(_NG, rowsv, tail_mask)
        pltpu.async_copy(rowsv.at[pl.ds(0, _CH)], acc.at[dstv], sems,
                         add=True)

    def dchunk(k, carry):
        j = k * 2
        chunk(j, *bufs[0])

        @pl.when(j + 1 < _NCHUNK)
        def _odd():
            chunk(j + 1, *bufs[1])

        return carry

    lax.fori_loop(0, (_NCHUNK + 1) // 2, dchunk, 0)
    for dstv, rowsv, sems in bufs:
        pltpu.make_async_copy(
            rowsv.at[pl.ds(0, _CH)], acc.at[dstv], sems).wait()
    plsc.subcore_barrier()

    @pl.when(sid < 15)
    def _copy_out():
        r0 = sid * _RPT
        pltpu.sync_copy(acc.at[pl.ds(r0, _RPT)],
                        out_hbm.at[cid, pl.ds(r0, _RPT)])

    @pl.when(sid == 15)
    def _copy_last():
        r0 = 15 * _RPT
        pltpu.sync_copy(acc.at[pl.ds(r0, _RLAST)],
                        out_hbm.at[cid, pl.ds(r0, _RLAST)])


_edge2 = functools.partial(
    pl.kernel,
    out_type=jax.ShapeDtypeStruct((_NC, _N, 8), jnp.float32),
    mesh=plsc.VectorSubcoreMesh(core_axis_name="c", subcore_axis_name="s",
                                num_cores=_NC, num_subcores=_NS),
    compiler_params=pltpu.CompilerParams(
        use_tc_tiling_on_sc=False, needs_layout_passes=False),
    scratch_types=[
        pltpu.VMEM((_CH,), jnp.int32),
        pltpu.VMEM((_CH,), jnp.int32),
        pltpu.VMEM((_CH,), jnp.int32),
        pltpu.VMEM((_CH + 16, 8), jnp.float32),
        pltpu.VMEM((_CH + 16,), jnp.float32),
        pltpu.VMEM((_CH + 16, 8), jnp.float32),
        pltpu.VMEM((_CH + 16, 8), jnp.float32),
        pltpu.VMEM((11, 16), jnp.float32),
        pltpu.VMEM((_SZ, 8), jnp.float32),
        pltpu.VMEM((_SZ, 8), jnp.float32),
        pltpu.VMEM((_SZ, 8), jnp.float32),
        pltpu.VMEM((_SZ,), jnp.float32),
        pltpu.VMEM_SHARED((_N, 8), jnp.float32),
        pltpu.VMEM_SHARED((_N,), jnp.float32),
        pltpu.VMEM_SHARED((_N, 8), jnp.float32),
        pltpu.SemaphoreType.DMA,
        pltpu.SemaphoreType.DMA,
        pltpu.SemaphoreType.DMA,
        pltpu.SemaphoreType.DMA,
    ])(_edge2_body)


def _final_body(acc_ref, c2_ref, out_ref):
    a = acc_ref[0] + acc_ref[1]
    p2 = a[:, 0]
    t2 = a[:, 1:5]
    den = p2 + 1e-9
    m = c2_ref[1:5, :]
    c0 = c2_ref[5, :]
    b_out = c2_ref[6, :]
    v = (t2 / den[:, None]) @ m + (p2 / den)[:, None] * c0[None] + b_out[None]
    out_ref[...] = jnp.where(v > 0, v, jnp.exp(jnp.minimum(v, 0.0)) - 1.0)


def _run_final(acc2, c2):
    return pl.pallas_call(
        _final_body,
        grid=(_NBLK,),
        in_specs=[
            pl.BlockSpec((_NC, _BN, 8), lambda i: (0, i, 0)),
            pl.BlockSpec((8, 16), lambda i: (0, 0)),
        ],
        out_specs=pl.BlockSpec((_BN, 16), lambda i: (i, 0)),
        out_shape=jax.ShapeDtypeStruct((_N, 16), jnp.float32),
    )(acc2, c2)


@jax.jit
def kernel(X_in1, edge_index, metadata_in1, W1, b1, W2, b2):
    x = X_in1.reshape(_N, 1)
    src = edge_index[0]
    dst = edge_index[1]
    c1, c2, c3 = _run_hyper(x, metadata_in1, W1, b1, W2, b2)
    zeros = jnp.zeros((_N, 8), jnp.float32)
    acc1 = _edge1(src, dst, x.reshape(_N), c1, zeros)
    acc2 = _edge2(src, dst, acc1, c3, zeros)
    return _run_final(acc2, c2)


# speedup vs baseline: 311.4047x; 1.1203x over previous
import functools
import jax
import jax.numpy as jnp
from jax import lax
from jax.experimental import pallas as pl
from jax.experimental.pallas import tpu as pltpu
from jax.experimental.pallas import tpu_sc as plsc

_N = 50000
_E = 800000
_H = 4
_D = 16
_ALPHA = 0.2
_NC = 2
_NS = 16
_NW = _NC * _NS
_EPW = _E // _NW
_CH = 1000
_NCHUNK = _EPW // _CH
_NG = _CH // 16
_BN = 5000
_NBLK = _N // _BN
_RPT = 3128
_RLAST = _N - 15 * _RPT


def _leaky(v):
    return jnp.maximum(v, _ALPHA * v)


def _hyper_body(x_ref, md_ref, w1_ref, b1_ref,
                wvw_ref, wvb_ref, bfw_ref, bfb_ref, alw_ref, alb_ref,
                arw_ref, arb_ref, w3w_ref, w3b_ref, bow_ref, bob_ref,
                aolw_ref, aolb_ref, aorw_ref, aorb_ref,
                g4_ref, g64_ref, g16_ref, rrep_ref, g4c_ref, ones_ref,
                out_ref):
    h = jnp.tanh(md_ref[...] @ w1_ref[...] + b1_ref[...])
    wvf = h @ wvw_ref[...] + wvb_ref[...]
    bfcf = h @ bfw_ref[...] + bfb_ref[...]
    alf = h @ alw_ref[...] + alb_ref[...]
    arf = h @ arw_ref[...] + arb_ref[...]
    w3f = h @ w3w_ref[...] + w3b_ref[...]
    b_out = h @ bow_ref[...] + bob_ref[...]
    a_l = h @ aolw_ref[...] + aolb_ref[...]
    a_r = h @ aorw_ref[...] + aorb_ref[...]

    g4 = g4_ref[...]
    cl = (wvf * alf) @ g4
    cr = (wvf * arf) @ g4
    wvt = jnp.concatenate([wvf] * 16, axis=1)
    bft = jnp.concatenate([bfcf] * 16, axis=1)
    m_flat = (w3f * wvt) @ g64_ref[...]
    c0 = (w3f * bft) @ g16_ref[...]
    alr = a_l @ rrep_ref[...]
    arr = a_r @ rrep_ref[...]
    g4c = g4c_ref[...]
    ml = (m_flat * alr) @ g4c
    mr = (m_flat * arr) @ g4c
    ones16 = ones_ref[...]
    cl2 = (c0 * a_l) @ ones16
    cr2 = (c0 * a_r) @ ones16

    xv = x_ref[...]
    mx = jnp.max(xv)
    mn = jnp.min(xv)
    ub = jnp.maximum(mx * cl, mn * cl) + jnp.maximum(mx * cr, mn * cr)
    m1 = _leaky(ub)

    maxax = jnp.maximum(mx, -mn)
    ub2 = (maxax * (jnp.sum(jnp.abs(ml)) + jnp.sum(jnp.abs(mr)))
           + cl2[0, 0] + cr2[0, 0])
    m2b = jnp.full((1, 1), _leaky(ub2), jnp.float32)

    z = jnp.zeros((1, 4), jnp.float32)
    out_ref[...] = jnp.concatenate(
        [cl, cr, m1, z, ml, mr, cl2, cr2, m2b, z[:, 0:1],
         m_flat, c0, b_out, z], axis=1)


def _run_hyper(x, metadata, w1, b1, w2, b2):
    hd = _H * _D
    o1, o2, o3 = 4 * hd, 4 * hd + _D * hd, 4 * hd + _D * hd + _D
    pieces = [(0, hd), (hd, 2 * hd), (2 * hd, 3 * hd), (3 * hd, 4 * hd),
              (o1, o2), (o2, o3), (o3, o3 + _D), (o3 + _D, o3 + 2 * _D)]
    args = [x, metadata, w1, b1[None]]
    for lo, hi in pieces:
        args.append(w2[:, lo:hi])
        args.append(b2[lo:hi][None])
    i64 = jnp.arange(64)
    i1024 = jnp.arange(1024)
    i16 = jnp.arange(16)
    g4 = (i64[:, None] // 16 == jnp.arange(4)[None]).astype(jnp.float32)
    kh = (i1024 // 64) * 4 + (i1024 % 64) // 16
    g64 = (kh[:, None] == i64[None]).astype(jnp.float32)
    g16 = (i1024[:, None] // 64 == i16[None]).astype(jnp.float32)
    rrep = (i16[:, None] == i64[None] // 4).astype(jnp.float32)
    g4c = (i64[:, None] % 4 == jnp.arange(4)[None]).astype(jnp.float32)
    ones16 = jnp.ones((16, 1), jnp.float32)
    args += [g4, g64, g16, rrep, g4c, ones16]
    cf = pl.pallas_call(
        _hyper_body,
        out_shape=jax.ShapeDtypeStruct((1, 128), jnp.float32),
    )(*args)[0]
    c1 = jnp.repeat(cf[0:12], 16).reshape(12, 16)
    c3 = jnp.repeat(cf[16:27], 16).reshape(11, 16)
    m = cf[28:92].reshape(16, 4).T
    row0 = jnp.concatenate([cf[16:24], cf[24:26], jnp.zeros((6,), jnp.float32)])
    c2 = jnp.concatenate([row0[None], m, cf[92:108][None], cf[108:124][None],
                          jnp.zeros((1, 16), jnp.float32)], axis=0)
    return c1, c2, c3


def _edge1_body(src_hbm, dst_hbm, x_hbm, c1_hbm, zeros_hbm, out_hbm,
                srcS, dstS, dstc0, dstc1, xsv0, xsv1, xdv0, xdv1,
                rowsv0, rowsv1, cv, acc,
                semg0, semg1, semh0, semh1, sems0, sems1):
    cid = lax.axis_index("c")
    sid = lax.axis_index("s")
    wid = sid * _NC + cid

    pltpu.sync_copy(c1_hbm, cv)
    ebase = wid * _EPW
    pltpu.sync_copy(src_hbm.at[pl.ds(ebase, _EPW)], srcS)
    pltpu.sync_copy(dst_hbm.at[pl.ds(ebase, _EPW)], dstS.at[pl.ds(0, _EPW)])

    @pl.when(sid == 0)
    def _zero():
        pltpu.sync_copy(zeros_hbm, acc)

    plsc.subcore_barrier()

    lane = lax.iota(jnp.int32, 16)
    tail_mask = lane < (_CH - _NG * 16)
    consts = [(cv[h, :], cv[4 + h, :], cv[8 + h, :]) for h in range(_H)]
    hcols = [jnp.full((16,), h, jnp.int32) for h in range(2 * _H)]
    bufs = [(dstc0, xsv0, xdv0, rowsv0, semg0, semh0, sems0),
            (dstc1, xsv1, xdv1, rowsv1, semg1, semh1, sems1)]

    def start(j, b):
        dstc, xsv, xdv, rowsv, semg, semh, sems = bufs[b]
        off = j * _CH
        pltpu.async_copy(x_hbm.at[srcS.at[pl.ds(off, _CH)]],
                         xsv.at[pl.ds(0, _CH)], semg)
        pltpu.async_copy(x_hbm.at[dstS.at[pl.ds(off, _CH)]],
                         xdv.at[pl.ds(0, _CH)], semh)

    def group(g, xsv, xdv, rowsv, mask):
        row16 = g * 16 + lane
        xs = xsv[pl.ds(g * 16, 16)]
        xd = xdv[pl.ds(g * 16, 16)]
        for h in range(_H):
            clh, crh, m1h = consts[h]
            pre = xs * clh + xd * crh
            pz = jnp.exp(jnp.maximum(pre, _ALPHA * pre) - m1h)
            plsc.store_scatter(rowsv, [row16, hcols[h]], pz, mask=mask)
            plsc.store_scatter(rowsv, [row16, hcols[_H + h]], pz * xs,
                               mask=mask)

    def chunk(j, b):
        dstc, xsv, xdv, rowsv, semg, semh, sems = bufs[b]
        pltpu.make_async_copy(x_hbm.at[srcS.at[pl.ds(0, _CH)]],
                              xsv.at[pl.ds(0, _CH)], semg).wait()
        pltpu.make_async_copy(x_hbm.at[srcS.at[pl.ds(0, _CH)]],
                              xdv.at[pl.ds(0, _CH)], semh).wait()

        @pl.when(j + 1 < _NCHUNK)
        def _pref():
            start(j + 1, 1 - b)

        @pl.when(j >= 2)
        def _drain():
            pltpu.make_async_copy(
                rowsv.at[pl.ds(0, _CH)], acc.at[dstc], sems).wait()

        def body(g, c2):
            group(g, xsv, xdv, rowsv, None)
            return c2

        lax.fori_loop(0, _NG, body, 0, unroll=4)
        group(_NG, xsv, xdv, rowsv, tail_mask)

        def icopy(g, c2):
            dstc[pl.ds(g * 16, 16)] = dstS[pl.ds(j * _CH + g * 16, 16)]
            return c2

        lax.fori_loop(0, _NG, icopy, 0, unroll=4)
        trow = _NG * 16 + lane
        plsc.store_scatter(dstc, [jnp.minimum(trow, _CH - 1)],
                           dstS[pl.ds(j * _CH + _NG * 16, 16)],
                           mask=tail_mask)
        pltpu.async_copy(rowsv.at[pl.ds(0, _CH)], acc.at[dstc], sems,
                         add=True)

    start(0, 0)

    def dchunk(k, carry):
        j = k * 2
        chunk(j, 0)

        @pl.when(j + 1 < _NCHUNK)
        def _odd():
            chunk(j + 1, 1)

        return carry

    lax.fori_loop(0, (_NCHUNK + 1) // 2, dchunk, 0)
    for dstc, _xs, _xd, rowsv, _sg, _sh, sems in bufs:
        pltpu.make_async_copy(
            rowsv.at[pl.ds(0, _CH)], acc.at[dstc], sems).wait()
    plsc.subcore_barrier()

    @pl.when(sid < 15)
    def _copy_out():
        r0 = sid * _RPT
        pltpu.sync_copy(acc.at[pl.ds(r0, _RPT)],
                        out_hbm.at[cid, pl.ds(r0, _RPT)])

    @pl.when(sid == 15)
    def _copy_last():
        r0 = 15 * _RPT
        pltpu.sync_copy(acc.at[pl.ds(r0, _RLAST)],
                        out_hbm.at[cid, pl.ds(r0, _RLAST)])


_edge1 = functools.partial(
    pl.kernel,
    out_type=jax.ShapeDtypeStruct((_NC, _N, 8), jnp.float32),
    mesh=plsc.VectorSubcoreMesh(core_axis_name="c", subcore_axis_name="s",
                                num_cores=_NC, num_subcores=_NS),
    compiler_params=pltpu.CompilerParams(
        use_tc_tiling_on_sc=False, needs_layout_passes=False),
    scratch_types=[
        pltpu.VMEM((_EPW,), jnp.int32),
        pltpu.VMEM((_EPW + 16,), jnp.int32),
        pltpu.VMEM((_CH,), jnp.int32),
        pltpu.VMEM((_CH,), jnp.int32),
        pltpu.VMEM((_CH + 16,), jnp.float32),
        pltpu.VMEM((_CH + 16,), jnp.float32),
        pltpu.VMEM((_CH + 16,), jnp.float32),
        pltpu.VMEM((_CH + 16,), jnp.float32),
        pltpu.VMEM((_CH + 16, 8), jnp.float32),
        pltpu.VMEM((_CH + 16, 8), jnp.float32),
        pltpu.VMEM((12, 16), jnp.float32),
        pltpu.VMEM_SHARED((_N, 8), jnp.float32),
        pltpu.SemaphoreType.DMA,
        pltpu.SemaphoreType.DMA,
        pltpu.SemaphoreType.DMA,
        pltpu.SemaphoreType.DMA,
        pltpu.SemaphoreType.DMA,
        pltpu.SemaphoreType.DMA,
    ])(_edge1_body)


_SZ = 256
_NSUB = _RPT // _SZ


def _edge2_body(src_hbm, dst_hbm, acc1_hbm, c3_hbm, zeros_hbm, out_hbm,
                srcv0, srcv1, dstv0, dstv1, dstc0, dstc1, urows, erd,
                rowsv0, rowsv1, cv, a0v, a1v, ubuf, erbuf, ush, ersh, acc,
                sem1, sem2, semi0, semi1, semj0, semj1, sems0, sems1):
    cid = lax.axis_index("c")
    sid = lax.axis_index("s")
    wid = sid * _NC + cid

    pltpu.sync_copy(c3_hbm, cv)

    @pl.when(sid == 0)
    def _zero():
        pltpu.sync_copy(zeros_hbm, acc)

    lane = lax.iota(jnp.int32, 16)
    zero16 = jnp.zeros((16,), jnp.int32)
    hcols = [jnp.full((16,), h, jnp.int32) for h in range(8)]
    mlv = [cv[h, :] for h in range(_H)]
    mrv = [cv[_H + h, :] for h in range(_H)]
    cl2v = cv[8, :]
    cr2v = cv[9, :]
    m2v = cv[10, :]

    def p1_group(g, r0, mask):
        row16 = g * 16 + lane
        a0c = [plsc.load_gather(a0v, [row16, hcols[c]]) for c in range(8)]
        a1c = [plsc.load_gather(a1v, [row16, hcols[c]]) for c in range(8)]
        s1 = [(a0c[4 + h] + a1c[4 + h]) / (a0c[h] + a1c[h] + 1e-9)
              for h in range(_H)]
        el2 = s1[0] * mlv[0] + s1[1] * mlv[1] + s1[2] * mlv[2] \
            + s1[3] * mlv[3] + cl2v
        er2 = s1[0] * mrv[0] + s1[1] * mrv[1] + s1[2] * mrv[2] \
            + s1[3] * mrv[3] + cr2v
        plsc.store_scatter(ubuf, [row16, hcols[0]], el2, mask=mask)
        for h in range(_H):
            plsc.store_scatter(ubuf, [row16, hcols[1 + h]], s1[h], mask=mask)
        plsc.store_scatter(erbuf, [row16], er2, mask=mask)

    def p1_block(r0, sz):
        pltpu.sync_copy(acc1_hbm.at[0, pl.ds(r0, sz)], a0v.at[pl.ds(0, sz)])
        pltpu.sync_copy(acc1_hbm.at[1, pl.ds(r0, sz)], a1v.at[pl.ds(0, sz)])

        def body(g, c):
            p1_group(g, r0, None)
            return c

        lax.fori_loop(0, sz // 16, body, 0, unroll=2)
        if sz % 16:
            p1_group(sz // 16, r0, lane < (sz % 16))
        pltpu.sync_copy(ubuf.at[pl.ds(0, sz)], ush.at[pl.ds(r0, sz)])
        pltpu.sync_copy(erbuf.at[pl.ds(0, sz)], ersh.at[pl.ds(r0, sz)])

    r0 = sid * _RPT

    def p1_sub(k, c):
        p1_block(r0 + k * _SZ, _SZ)
        return c

    lax.fori_loop(0, _NSUB, p1_sub, 0)

    @pl.when(sid < 15)
    def _p1_rem():
        p1_block(r0 + _NSUB * _SZ, _RPT - _NSUB * _SZ)

    @pl.when(sid == 15)
    def _p1_rem_last():
        p1_block(r0 + _NSUB * _SZ, _RLAST - _NSUB * _SZ)

    plsc.subcore_barrier()

    tail_mask = lane < (_CH - _NG * 16)
    bufs = [(srcv0, dstv0, dstc0, rowsv0, semi0, semj0, sems0),
            (srcv1, dstv1, dstc1, rowsv1, semi1, semj1, sems1)]

    def zrow(r, c2):
        ridx = r * 2 + (lane >> 3)
        ccol = 5 + (lane & 7)
        zmask = (lane & 7) < 3
        zv = jnp.zeros((16,), jnp.float32)
        plsc.store_scatter(rowsv0, [ridx, ccol], zv, mask=zmask)
        plsc.store_scatter(rowsv1, [ridx, ccol], zv, mask=zmask)
        return c2

    lax.fori_loop(0, (_CH + 16) // 2, zrow, 0, unroll=4)

    def group(g, rowsv, mask):
        row16 = g * 16 + lane
        el2s = plsc.load_gather(urows, [row16, zero16])
        erdg = erd[pl.ds(g * 16, 16)]
        pre = el2s + erdg
        p2 = jnp.exp(jnp.maximum(pre, _ALPHA * pre) - m2v)
        plsc.store_scatter(rowsv, [row16, hcols[0]], p2, mask=mask)
        for h in range(1, 5):
            vals = plsc.load_gather(urows, [row16, hcols[h]])
            plsc.store_scatter(rowsv, [row16, hcols[h]], p2 * vals, mask=mask)

    def start(j, b):
        srcv, dstv, dstc, rowsv, semi, semj, sems = bufs[b]
        base = wid * _EPW + j * _CH
        pltpu.async_copy(src_hbm.at[pl.ds(base, _CH)], srcv, semi)
        pltpu.async_copy(dst_hbm.at[pl.ds(base, _CH)],
                         dstv.at[pl.ds(0, _CH)], semj)

    def chunk(j, b):
        srcv, dstv, dstc, rowsv, semi, semj, sems = bufs[b]
        base = wid * _EPW + j * _CH
        pltpu.make_async_copy(src_hbm.at[pl.ds(base, _CH)], srcv,
                              semi).wait()
        pltpu.make_async_copy(dst_hbm.at[pl.ds(base, _CH)],
                              dstv.at[pl.ds(0, _CH)], semj).wait()

        @pl.when(j + 1 < _NCHUNK)
        def _pref():
            start(j + 1, 1 - b)

        cp1 = pltpu.async_copy(ush.at[srcv], urows.at[pl.ds(0, _CH)], sem1)
        cp2 = pltpu.async_copy(ersh.at[dstv.at[pl.ds(0, _CH)]],
                               erd.at[pl.ds(0, _CH)], sem2)
        cp1.wait()
        cp2.wait()

        @pl.when(j >= 2)
        def _drain():
            pltpu.make_async_copy(
                rowsv.at[pl.ds(0, _CH)], acc.at[dstc], sems).wait()

        def body(g, c2):
            group(g, rowsv, None)
            return c2

        lax.fori_loop(0, _NG, body, 0, unroll=4)
        group(_NG, rowsv, tail_mask)

        def icopy(g, c2):
            dstc[pl.ds(g * 16, 16)] = dstv[pl.ds(g * 16, 16)]
            return c2

        lax.fori_loop(0, _NG, icopy, 0, unroll=4)
        trow = _NG * 16 + lane
        plsc.store_scatter(dstc, [jnp.minimum(trow, _CH - 1)],
                           dstv[pl.ds(_NG * 16, 16)], mask=tail_mask)
        pltpu.async_copy(rowsv.at[pl.ds(0, _CH)], acc.at[dstc], sems,
                         add=True)

    start(0, 0)

    def dchunk(k, carry):
        j = k * 2
        chunk(j, 0)

        @pl.when(j + 1 < _NCHUNK)
        def _odd():
            chunk(j + 1, 1)

        return carry

    lax.fori_loop(0, (_NCHUNK + 1) // 2, dchunk, 0)
    for _sv, _dv, dstc, rowsv, _si, _sj, sems in bufs:
        pltpu.make_async_copy(
            rowsv.at[pl.ds(0, _CH)], acc.at[dstc], sems).wait()
    plsc.subcore_barrier()

    @pl.when(sid < 15)
    def _copy_out():
        r0 = sid * _RPT
        pltpu.sync_copy(acc.at[pl.ds(r0, _RPT)],
                        out_hbm.at[cid, pl.ds(r0, _RPT)])

    @pl.when(sid == 15)
    def _copy_last():
        r0 = 15 * _RPT
        pltpu.sync_copy(acc.at[pl.ds(r0, _RLAST)],
                        out_hbm.at[cid, pl.ds(r0, _RLAST)])


_edge2 = functools.partial(
    pl.kernel,
    out_type=jax.ShapeDtypeStruct((_NC, _N, 8), jnp.float32),
    mesh=plsc.VectorSubcoreMesh(core_axis_name="c", subcore_axis_name="s",
                                num_cores=_NC, num_subcores=_NS),
    compiler_params=pltpu.CompilerParams(
        use_tc_tiling_on_sc=False, needs_layout_passes=False),
    scratch_types=[
        pltpu.VMEM((_CH,), jnp.int32),
        pltpu.VMEM((_CH,), jnp.int32),
        pltpu.VMEM((_CH + 16,), jnp.int32),
        pltpu.VMEM((_CH + 16,), jnp.int32),
        pltpu.VMEM((_CH,), jnp.int32),
        pltpu.VMEM((_CH,), jnp.int32),
        pltpu.VMEM((_CH + 16, 8), jnp.float32),
        pltpu.VMEM((_CH + 16,), jnp.float32),
        pltpu.VMEM((_CH + 16, 8), jnp.float32),
        pltpu.VMEM((_CH + 16, 8), jnp.float32),
        pltpu.VMEM((11, 16), jnp.float32),
        pltpu.VMEM((_SZ, 8), jnp.float32),
        pltpu.VMEM((_SZ, 8), jnp.float32),
        pltpu.VMEM((_SZ, 8), jnp.float32),
        pltpu.VMEM((_SZ,), jnp.float32),
        pltpu.VMEM_SHARED((_N, 8), jnp.float32),
        pltpu.VMEM_SHARED((_N,), jnp.float32),
        pltpu.VMEM_SHARED((_N, 8), jnp.float32),
        pltpu.SemaphoreType.DMA,
        pltpu.SemaphoreType.DMA,
        pltpu.SemaphoreType.DMA,
        pltpu.SemaphoreType.DMA,
        pltpu.SemaphoreType.DMA,
        pltpu.SemaphoreType.DMA,
        pltpu.SemaphoreType.DMA,
        pltpu.SemaphoreType.DMA,
    ])(_edge2_body)


def _final_body(acc_ref, c2_ref, out_ref):
    a = acc_ref[0] + acc_ref[1]
    p2 = a[:, 0]
    t2 = a[:, 1:5]
    den = p2 + 1e-9
    m = c2_ref[1:5, :]
    c0 = c2_ref[5, :]
    b_out = c2_ref[6, :]
    v = (t2 / den[:, None]) @ m + (p2 / den)[:, None] * c0[None] + b_out[None]
    out_ref[...] = jnp.where(v > 0, v, jnp.exp(jnp.minimum(v, 0.0)) - 1.0)


def _run_final(acc2, c2):
    return pl.pallas_call(
        _final_body,
        grid=(_NBLK,),
        in_specs=[
            pl.BlockSpec((_NC, _BN, 8), lambda i: (0, i, 0)),
            pl.BlockSpec((8, 16), lambda i: (0, 0)),
        ],
        out_specs=pl.BlockSpec((_BN, 16), lambda i: (i, 0)),
        out_shape=jax.ShapeDtypeStruct((_N, 16), jnp.float32),
    )(acc2, c2)


@jax.jit
def kernel(X_in1, edge_index, metadata_in1, W1, b1, W2, b2):
    x = X_in1.reshape(_N, 1)
    src = edge_index[0]
    dst = edge_index[1]
    c1, c2, c3 = _run_hyper(x, metadata_in1, W1, b1, W2, b2)
    zeros = jnp.zeros((_N, 8), jnp.float32)
    acc1 = _edge1(src, dst, x.reshape(_N), c1, zeros)
    acc2 = _edge2(src, dst, acc1, c3, zeros)
    return _run_final(acc2, c2)


# speedup vs baseline: 322.3220x; 1.0351x over previous
import functools
import jax
import jax.numpy as jnp
from jax import lax
from jax.experimental import pallas as pl
from jax.experimental.pallas import tpu as pltpu
from jax.experimental.pallas import tpu_sc as plsc

_N = 50000
_E = 800000
_H = 4
_D = 16
_ALPHA = 0.2
_NC = 2
_NS = 16
_NW = _NC * _NS
_EPW = _E // _NW
_CH = 1000
_NCHUNK = _EPW // _CH
_NG = _CH // 16
_BN = 10000
_NBLK = _N // _BN
_RPT = 3128
_RLAST = _N - 15 * _RPT


def _leaky(v):
    return jnp.maximum(v, _ALPHA * v)


def _hyper_body(x_ref, md_ref, w1_ref, b1_ref,
                wvw_ref, wvb_ref, bfw_ref, bfb_ref, alw_ref, alb_ref,
                arw_ref, arb_ref, w3w_ref, w3b_ref, bow_ref, bob_ref,
                aolw_ref, aolb_ref, aorw_ref, aorb_ref,
                g4_ref, g64_ref, g16_ref, rrep_ref, g4c_ref, ones_ref,
                out_ref):
    h = jnp.tanh(md_ref[...] @ w1_ref[...] + b1_ref[...])
    wvf = h @ wvw_ref[...] + wvb_ref[...]
    bfcf = h @ bfw_ref[...] + bfb_ref[...]
    alf = h @ alw_ref[...] + alb_ref[...]
    arf = h @ arw_ref[...] + arb_ref[...]
    w3f = h @ w3w_ref[...] + w3b_ref[...]
    b_out = h @ bow_ref[...] + bob_ref[...]
    a_l = h @ aolw_ref[...] + aolb_ref[...]
    a_r = h @ aorw_ref[...] + aorb_ref[...]

    g4 = g4_ref[...]
    cl = (wvf * alf) @ g4
    cr = (wvf * arf) @ g4
    wvt = jnp.concatenate([wvf] * 16, axis=1)
    bft = jnp.concatenate([bfcf] * 16, axis=1)
    m_flat = (w3f * wvt) @ g64_ref[...]
    c0 = (w3f * bft) @ g16_ref[...]
    alr = a_l @ rrep_ref[...]
    arr = a_r @ rrep_ref[...]
    g4c = g4c_ref[...]
    ml = (m_flat * alr) @ g4c
    mr = (m_flat * arr) @ g4c
    ones16 = ones_ref[...]
    cl2 = (c0 * a_l) @ ones16
    cr2 = (c0 * a_r) @ ones16

    xv = x_ref[...]
    mx = jnp.max(xv)
    mn = jnp.min(xv)
    ub = jnp.maximum(mx * cl, mn * cl) + jnp.maximum(mx * cr, mn * cr)
    m1 = _leaky(ub)

    maxax = jnp.maximum(mx, -mn)
    ub2 = (maxax * (jnp.sum(jnp.abs(ml)) + jnp.sum(jnp.abs(mr)))
           + cl2[0, 0] + cr2[0, 0])
    m2b = jnp.full((1, 1), _leaky(ub2), jnp.float32)

    z = jnp.zeros((1, 4), jnp.float32)
    out_ref[...] = jnp.concatenate(
        [cl, cr, m1, z, ml, mr, cl2, cr2, m2b, z[:, 0:1],
         m_flat, c0, b_out, z], axis=1)


def _run_hyper(x, metadata, w1, b1, w2, b2):
    hd = _H * _D
    o1, o2, o3 = 4 * hd, 4 * hd + _D * hd, 4 * hd + _D * hd + _D
    pieces = [(0, hd), (hd, 2 * hd), (2 * hd, 3 * hd), (3 * hd, 4 * hd),
              (o1, o2), (o2, o3), (o3, o3 + _D), (o3 + _D, o3 + 2 * _D)]
    args = [x.reshape(400, 125), metadata, w1, b1[None]]
    for lo, hi in pieces:
        args.append(w2[:, lo:hi])
        args.append(b2[lo:hi][None])
    i64 = jnp.arange(64)
    i1024 = jnp.arange(1024)
    i16 = jnp.arange(16)
    g4 = (i64[:, None] // 16 == jnp.arange(4)[None]).astype(jnp.float32)
    kh = (i1024 // 64) * 4 + (i1024 % 64) // 16
    g64 = (kh[:, None] == i64[None]).astype(jnp.float32)
    g16 = (i1024[:, None] // 64 == i16[None]).astype(jnp.float32)
    rrep = (i16[:, None] == i64[None] // 4).astype(jnp.float32)
    g4c = (i64[:, None] % 4 == jnp.arange(4)[None]).astype(jnp.float32)
    ones16 = jnp.ones((16, 1), jnp.float32)
    args += [g4, g64, g16, rrep, g4c, ones16]
    cf = pl.pallas_call(
        _hyper_body,
        out_shape=jax.ShapeDtypeStruct((1, 128), jnp.float32),
    )(*args)[0]
    c1 = jnp.repeat(cf[0:12], 16).reshape(12, 16)
    c3 = jnp.repeat(cf[16:27], 16).reshape(11, 16)
    m = cf[28:92].reshape(16, 4).T
    row0 = jnp.concatenate([cf[16:24], cf[24:26], jnp.zeros((6,), jnp.float32)])
    c2 = jnp.concatenate([row0[None], m, cf[92:108][None], cf[108:124][None],
                          jnp.zeros((1, 16), jnp.float32)], axis=0)
    return c1, c2, c3


def _edge1_body(src_hbm, dst_hbm, x_hbm, c1_hbm, zeros_hbm, out_hbm,
                srcS, dstS, dstc0, dstc1, xsv0, xsv1, xdv0, xdv1,
                rowsv0, rowsv1, cv, acc,
                semg0, semg1, semh0, semh1, sems0, sems1):
    cid = lax.axis_index("c")
    sid = lax.axis_index("s")
    wid = sid * _NC + cid

    pltpu.sync_copy(c1_hbm, cv)
    ebase = wid * _EPW
    pltpu.sync_copy(src_hbm.at[pl.ds(ebase, _EPW)], srcS)
    pltpu.sync_copy(dst_hbm.at[pl.ds(ebase, _EPW)], dstS.at[pl.ds(0, _EPW)])

    @pl.when(sid == 0)
    def _zero():
        pltpu.sync_copy(zeros_hbm, acc)

    plsc.subcore_barrier()

    lane = lax.iota(jnp.int32, 16)
    tail_mask = lane < (_CH - _NG * 16)
    consts = [(cv[h, :], cv[4 + h, :], cv[8 + h, :]) for h in range(_H)]
    hcols = [jnp.full((16,), h, jnp.int32) for h in range(2 * _H)]
    bufs = [(dstc0, xsv0, xdv0, rowsv0, semg0, semh0, sems0),
            (dstc1, xsv1, xdv1, rowsv1, semg1, semh1, sems1)]

    def start(j, b):
        dstc, xsv, xdv, rowsv, semg, semh, sems = bufs[b]
        off = j * _CH
        pltpu.async_copy(x_hbm.at[srcS.at[pl.ds(off, _CH)]],
                         xsv.at[pl.ds(0, _CH)], semg)
        pltpu.async_copy(x_hbm.at[dstS.at[pl.ds(off, _CH)]],
                         xdv.at[pl.ds(0, _CH)], semh)

    def group(g, xsv, xdv, rowsv, mask):
        row16 = g * 16 + lane
        xs = xsv[pl.ds(g * 16, 16)]
        xd = xdv[pl.ds(g * 16, 16)]
        for h in range(_H):
            clh, crh, m1h = consts[h]
            pre = xs * clh + xd * crh
            pz = jnp.exp(jnp.maximum(pre, _ALPHA * pre) - m1h)
            plsc.store_scatter(rowsv, [row16, hcols[h]], pz, mask=mask)
            plsc.store_scatter(rowsv, [row16, hcols[_H + h]], pz * xs,
                               mask=mask)

    def chunk(j, b):
        dstc, xsv, xdv, rowsv, semg, semh, sems = bufs[b]
        pltpu.make_async_copy(x_hbm.at[srcS.at[pl.ds(0, _CH)]],
                              xsv.at[pl.ds(0, _CH)], semg).wait()
        pltpu.make_async_copy(x_hbm.at[srcS.at[pl.ds(0, _CH)]],
                              xdv.at[pl.ds(0, _CH)], semh).wait()

        @pl.when(j + 1 < _NCHUNK)
        def _pref():
            start(j + 1, 1 - b)

        @pl.when(j >= 2)
        def _drain():
            pltpu.make_async_copy(
                rowsv.at[pl.ds(0, _CH)], acc.at[dstc], sems).wait()

        def body(g, c2):
            group(g, xsv, xdv, rowsv, None)
            return c2

        lax.fori_loop(0, _NG, body, 0, unroll=4)
        group(_NG, xsv, xdv, rowsv, tail_mask)

        def icopy(g, c2):
            dstc[pl.ds(g * 16, 16)] = dstS[pl.ds(j * _CH + g * 16, 16)]
            return c2

        lax.fori_loop(0, _NG, icopy, 0, unroll=4)
        trow = _NG * 16 + lane
        plsc.store_scatter(dstc, [jnp.minimum(trow, _CH - 1)],
                           dstS[pl.ds(j * _CH + _NG * 16, 16)],
                           mask=tail_mask)
        pltpu.async_copy(rowsv.at[pl.ds(0, _CH)], acc.at[dstc], sems,
                         add=True)

    start(0, 0)

    def dchunk(k, carry):
        j = k * 2
        chunk(j, 0)

        @pl.when(j + 1 < _NCHUNK)
        def _odd():
            chunk(j + 1, 1)

        return carry

    lax.fori_loop(0, (_NCHUNK + 1) // 2, dchunk, 0)
    for dstc, _xs, _xd, rowsv, _sg, _sh, sems in bufs:
        pltpu.make_async_copy(
            rowsv.at[pl.ds(0, _CH)], acc.at[dstc], sems).wait()
    plsc.subcore_barrier()

    @pl.when(sid < 15)
    def _copy_out():
        r0 = sid * _RPT
        pltpu.sync_copy(acc.at[pl.ds(r0, _RPT)],
                        out_hbm.at[cid, pl.ds(r0, _RPT)])

    @pl.when(sid == 15)
    def _copy_last():
        r0 = 15 * _RPT
        pltpu.sync_copy(acc.at[pl.ds(r0, _RLAST)],
                        out_hbm.at[cid, pl.ds(r0, _RLAST)])


_edge1 = functools.partial(
    pl.kernel,
    out_type=jax.ShapeDtypeStruct((_NC, _N, 8), jnp.float32),
    mesh=plsc.VectorSubcoreMesh(core_axis_name="c", subcore_axis_name="s",
                                num_cores=_NC, num_subcores=_NS),
    compiler_params=pltpu.CompilerParams(
        use_tc_tiling_on_sc=False, needs_layout_passes=False),
    scratch_types=[
        pltpu.VMEM((_EPW,), jnp.int32),
        pltpu.VMEM((_EPW + 16,), jnp.int32),
        pltpu.VMEM((_CH,), jnp.int32),
        pltpu.VMEM((_CH,), jnp.int32),
        pltpu.VMEM((_CH + 16,), jnp.float32),
        pltpu.VMEM((_CH + 16,), jnp.float32),
        pltpu.VMEM((_CH + 16,), jnp.float32),
        pltpu.VMEM((_CH + 16,), jnp.float32),
        pltpu.VMEM((_CH + 16, 8), jnp.float32),
        pltpu.VMEM((_CH + 16, 8), jnp.float32),
        pltpu.VMEM((12, 16), jnp.float32),
        pltpu.VMEM_SHARED((_N, 8), jnp.float32),
        pltpu.SemaphoreType.DMA,
        pltpu.SemaphoreType.DMA,
        pltpu.SemaphoreType.DMA,
        pltpu.SemaphoreType.DMA,
        pltpu.SemaphoreType.DMA,
        pltpu.SemaphoreType.DMA,
    ])(_edge1_body)


_SZ = 256
_NSUB = _RPT // _SZ


def _edge2_body(src_hbm, dst_hbm, acc1_hbm, c3_hbm, zeros_hbm, out_hbm,
                srcv0, srcv1, dstv0, dstv1, dstc0, dstc1, urows, erd,
                rowsv0, rowsv1, cv, a0v, a1v, ubuf, erbuf, ush, ersh, acc,
                sem1, sem2, semi0, semi1, semj0, semj1, sems0, sems1):
    cid = lax.axis_index("c")
    sid = lax.axis_index("s")
    wid = sid * _NC + cid

    pltpu.sync_copy(c3_hbm, cv)

    @pl.when(sid == 0)
    def _zero():
        pltpu.sync_copy(zeros_hbm, acc)

    lane = lax.iota(jnp.int32, 16)
    zero16 = jnp.zeros((16,), jnp.int32)
    hcols = [jnp.full((16,), h, jnp.int32) for h in range(8)]
    mlv = [cv[h, :] for h in range(_H)]
    mrv = [cv[_H + h, :] for h in range(_H)]
    cl2v = cv[8, :]
    cr2v = cv[9, :]
    m2v = cv[10, :]

    def p1_group(g, r0, mask):
        row16 = g * 16 + lane
        a0c = [plsc.load_gather(a0v, [row16, hcols[c]]) for c in range(8)]
        a1c = [plsc.load_gather(a1v, [row16, hcols[c]]) for c in range(8)]
        s1 = [(a0c[4 + h] + a1c[4 + h]) / (a0c[h] + a1c[h] + 1e-9)
              for h in range(_H)]
        el2 = s1[0] * mlv[0] + s1[1] * mlv[1] + s1[2] * mlv[2] \
            + s1[3] * mlv[3] + cl2v
        er2 = s1[0] * mrv[0] + s1[1] * mrv[1] + s1[2] * mrv[2] \
            + s1[3] * mrv[3] + cr2v
        plsc.store_scatter(ubuf, [row16, hcols[0]], el2, mask=mask)
        for h in range(_H):
            plsc.store_scatter(ubuf, [row16, hcols[1 + h]], s1[h], mask=mask)
        plsc.store_scatter(erbuf, [row16], er2, mask=mask)

    def p1_block(r0, sz):
        pltpu.sync_copy(acc1_hbm.at[0, pl.ds(r0, sz)], a0v.at[pl.ds(0, sz)])
        pltpu.sync_copy(acc1_hbm.at[1, pl.ds(r0, sz)], a1v.at[pl.ds(0, sz)])

        def body(g, c):
            p1_group(g, r0, None)
            return c

        lax.fori_loop(0, sz // 16, body, 0, unroll=2)
        if sz % 16:
            p1_group(sz // 16, r0, lane < (sz % 16))
        pltpu.sync_copy(ubuf.at[pl.ds(0, sz)], ush.at[pl.ds(r0, sz)])
        pltpu.sync_copy(erbuf.at[pl.ds(0, sz)], ersh.at[pl.ds(r0, sz)])

    r0 = sid * _RPT

    def p1_sub(k, c):
        p1_block(r0 + k * _SZ, _SZ)
        return c

    lax.fori_loop(0, _NSUB, p1_sub, 0)

    @pl.when(sid < 15)
    def _p1_rem():
        p1_block(r0 + _NSUB * _SZ, _RPT - _NSUB * _SZ)

    @pl.when(sid == 15)
    def _p1_rem_last():
        p1_block(r0 + _NSUB * _SZ, _RLAST - _NSUB * _SZ)

    plsc.subcore_barrier()

    tail_mask = lane < (_CH - _NG * 16)
    bufs = [(srcv0, dstv0, dstc0, rowsv0, semi0, semj0, sems0),
            (srcv1, dstv1, dstc1, rowsv1, semi1, semj1, sems1)]

    def zrow(r, c2):
        ridx = r * 2 + (lane >> 3)
        ccol = 5 + (lane & 7)
        zmask = (lane & 7) < 3
        zv = jnp.zeros((16,), jnp.float32)
        plsc.store_scatter(rowsv0, [ridx, ccol], zv, mask=zmask)
        plsc.store_scatter(rowsv1, [ridx, ccol], zv, mask=zmask)
        return c2

    lax.fori_loop(0, (_CH + 16) // 2, zrow, 0, unroll=4)

    def group(g, rowsv, mask):
        row16 = g * 16 + lane
        el2s = plsc.load_gather(urows, [row16, zero16])
        erdg = erd[pl.ds(g * 16, 16)]
        pre = el2s + erdg
        p2 = jnp.exp(jnp.maximum(pre, _ALPHA * pre) - m2v)
        plsc.store_scatter(rowsv, [row16, hcols[0]], p2, mask=mask)
        for h in range(1, 5):
            vals = plsc.load_gather(urows, [row16, hcols[h]])
            plsc.store_scatter(rowsv, [row16, hcols[h]], p2 * vals, mask=mask)

    def start(j, b):
        srcv, dstv, dstc, rowsv, semi, semj, sems = bufs[b]
        base = wid * _EPW + j * _CH
        pltpu.async_copy(src_hbm.at[pl.ds(base, _CH)], srcv, semi)
        pltpu.async_copy(dst_hbm.at[pl.ds(base, _CH)],
                         dstv.at[pl.ds(0, _CH)], semj)

    def chunk(j, b):
        srcv, dstv, dstc, rowsv, semi, semj, sems = bufs[b]
        base = wid * _EPW + j * _CH
        pltpu.make_async_copy(src_hbm.at[pl.ds(base, _CH)], srcv,
                              semi).wait()
        pltpu.make_async_copy(dst_hbm.at[pl.ds(base, _CH)],
                              dstv.at[pl.ds(0, _CH)], semj).wait()

        @pl.when(j + 1 < _NCHUNK)
        def _pref():
            start(j + 1, 1 - b)

        cp1 = pltpu.async_copy(ush.at[srcv], urows.at[pl.ds(0, _CH)], sem1)
        cp2 = pltpu.async_copy(ersh.at[dstv.at[pl.ds(0, _CH)]],
                               erd.at[pl.ds(0, _CH)], sem2)
        cp1.wait()
        cp2.wait()

        @pl.when(j >= 2)
        def _drain():
            pltpu.make_async_copy(
                rowsv.at[pl.ds(0, _CH)], acc.at[dstc], sems).wait()

        def body(g, c2):
            group(g, rowsv, None)
            return c2

        lax.fori_loop(0, _NG, body, 0, unroll=4)
        group(_NG, rowsv, tail_mask)

        def icopy(g, c2):
            dstc[pl.ds(g * 16, 16)] = dstv[pl.ds(g * 16, 16)]
            return c2

        lax.fori_loop(0, _NG, icopy, 0, unroll=4)
        trow = _NG * 16 + lane
        plsc.store_scatter(dstc, [jnp.minimum(trow, _CH - 1)],
                           dstv[pl.ds(_NG * 16, 16)], mask=tail_mask)
        pltpu.async_copy(rowsv.at[pl.ds(0, _CH)], acc.at[dstc], sems,
                         add=True)

    start(0, 0)

    def dchunk(k, carry):
        j = k * 2
        chunk(j, 0)

        @pl.when(j + 1 < _NCHUNK)
        def _odd():
            chunk(j + 1, 1)

        return carry

    lax.fori_loop(0, (_NCHUNK + 1) // 2, dchunk, 0)
    for _sv, _dv, dstc, rowsv, _si, _sj, sems in bufs:
        pltpu.make_async_copy(
            rowsv.at[pl.ds(0, _CH)], acc.at[dstc], sems).wait()
    plsc.subcore_barrier()

    @pl.when(sid < 15)
    def _copy_out():
        r0 = sid * _RPT
        pltpu.sync_copy(acc.at[pl.ds(r0, _RPT)],
                        out_hbm.at[cid, pl.ds(r0, _RPT)])

    @pl.when(sid == 15)
    def _copy_last():
        r0 = 15 * _RPT
        pltpu.sync_copy(acc.at[pl.ds(r0, _RLAST)],
                        out_hbm.at[cid, pl.ds(r0, _RLAST)])


_edge2 = functools.partial(
    pl.kernel,
    out_type=jax.ShapeDtypeStruct((_NC, _N, 8), jnp.float32),
    mesh=plsc.VectorSubcoreMesh(core_axis_name="c", subcore_axis_name="s",
                                num_cores=_NC, num_subcores=_NS),
    compiler_params=pltpu.CompilerParams(
        use_tc_tiling_on_sc=False, needs_layout_passes=False),
    scratch_types=[
        pltpu.VMEM((_CH,), jnp.int32),
        pltpu.VMEM((_CH,), jnp.int32),
        pltpu.VMEM((_CH + 16,), jnp.int32),
        pltpu.VMEM((_CH + 16,), jnp.int32),
        pltpu.VMEM((_CH,), jnp.int32),
        pltpu.VMEM((_CH,), jnp.int32),
        pltpu.VMEM((_CH + 16, 8), jnp.float32),
        pltpu.VMEM((_CH + 16,), jnp.float32),
        pltpu.VMEM((_CH + 16, 8), jnp.float32),
        pltpu.VMEM((_CH + 16, 8), jnp.float32),
        pltpu.VMEM((11, 16), jnp.float32),
        pltpu.VMEM((_SZ, 8), jnp.float32),
        pltpu.VMEM((_SZ, 8), jnp.float32),
        pltpu.VMEM((_SZ, 8), jnp.float32),
        pltpu.VMEM((_SZ,), jnp.float32),
        pltpu.VMEM_SHARED((_N, 8), jnp.float32),
        pltpu.VMEM_SHARED((_N,), jnp.float32),
        pltpu.VMEM_SHARED((_N, 8), jnp.float32),
        pltpu.SemaphoreType.DMA,
        pltpu.SemaphoreType.DMA,
        pltpu.SemaphoreType.DMA,
        pltpu.SemaphoreType.DMA,
        pltpu.SemaphoreType.DMA,
        pltpu.SemaphoreType.DMA,
        pltpu.SemaphoreType.DMA,
        pltpu.SemaphoreType.DMA,
    ])(_edge2_body)


def _final_body(acc_ref, c2_ref, out_ref):
    a = acc_ref[0] + acc_ref[1]
    p2 = a[:, 0]
    t2 = a[:, 1:5]
    den = p2 + 1e-9
    m = c2_ref[1:5, :]
    c0 = c2_ref[5, :]
    b_out = c2_ref[6, :]
    v = (t2 / den[:, None]) @ m + (p2 / den)[:, None] * c0[None] + b_out[None]
    out_ref[...] = jnp.where(v > 0, v, jnp.exp(jnp.minimum(v, 0.0)) - 1.0)


def _run_final(acc2, c2):
    return pl.pallas_call(
        _final_body,
        grid=(_NBLK,),
        in_specs=[
            pl.BlockSpec((_NC, _BN, 8), lambda i: (0, i, 0)),
            pl.BlockSpec((8, 16), lambda i: (0, 0)),
        ],
        out_specs=pl.BlockSpec((_BN, 16), lambda i: (i, 0)),
        out_shape=jax.ShapeDtypeStruct((_N, 16), jnp.float32),
    )(acc2, c2)


@jax.jit
def kernel(X_in1, edge_index, metadata_in1, W1, b1, W2, b2):
    x = X_in1.reshape(_N, 1)
    src = edge_index[0]
    dst = edge_index[1]
    c1, c2, c3 = _run_hyper(x, metadata_in1, W1, b1, W2, b2)
    zeros = jnp.zeros((_N, 8), jnp.float32)
    acc1 = _edge1(src, dst, x.reshape(_N), c1, zeros)
    acc2 = _edge2(src, dst, acc1, c3, zeros)
    return _run_final(acc2, c2)
